# Initial kernel scaffold; baseline (speedup 1.0000x reference)
import jax
import jax.numpy as jnp
from jax.experimental import pallas as pl


def kernel(h, norm, e_h, s_h, edge_index, text_W, text_b, inv_W, inv_b, rel_W, rel_b, bias_v):
    raise NotImplementedError("write your pallas kernel here")


# speedup vs baseline: 4.8435x; 4.8435x over previous
import functools

import jax
import jax.numpy as jnp
from jax import lax
from jax.experimental import pallas as pl
from jax.experimental.pallas import tpu as pltpu
from jax.experimental.pallas import tpu_sc as plsc

F32 = jnp.float32

NC = 2
NS = 16
C = 80


def _sc_mesh():
    return plsc.VectorSubcoreMesh(core_axis_name="c", subcore_axis_name="s")


def _row_copy(src, dst, sid, n):
    per = (-(-n // NS) + 7) // 8 * 8
    last = n - (NS - 1) * per
    assert last > 0 and last % 8 == 0

    @pl.when(sid < NS - 1)
    def _():
        st = pl.multiple_of(sid * per, 8)
        pltpu.sync_copy(src.at[pl.ds(st, per), :], dst.at[pl.ds(st, per), :])

    @pl.when(sid == NS - 1)
    def _():
        st = (NS - 1) * per
        pltpu.sync_copy(src.at[pl.ds(st, last), :], dst.at[pl.ds(st, last), :])


def _make_k1(n, e):
    ept = e // NS
    nch = ept // C

    @functools.partial(
        pl.kernel,
        out_type=[
            jax.ShapeDtypeStruct((n, 128), F32),
            jax.ShapeDtypeStruct((n, 128), F32),
            jax.ShapeDtypeStruct((n,), F32),
        ],
        mesh=_sc_mesh(),
        scratch_types=[
            pltpu.VMEM_SHARED((n, 128), F32),
            pltpu.VMEM_SHARED((n,), F32),
            pltpu.VMEM((1, C), jnp.int32),
            pltpu.VMEM((C, 128), F32),
            pltpu.VMEM((C,), F32),
        ],
    )
    def k1(dst_r, eh_r, sh_r, z2_r, z1_r, out_se, out_ss, out_dg,
           acc, dacc, idxb, rows, ones_v):
        cid = lax.axis_index("c")
        sid = lax.axis_index("s")
        for t in range(C // 16):
            ones_v[pl.ds(16 * t, 16)] = jnp.full((16,), 1.0, F32)
        _row_copy(z2_r, acc, sid, n)

        @pl.when(jnp.logical_and(sid == 0, cid == 0))
        def _():
            pltpu.sync_copy(z1_r, dacc)

        plsc.subcore_barrier()

        def chunk(i, arr_r, do_deg):
            off = pl.multiple_of(sid * ept + i * C, 8)
            pltpu.sync_copy(dst_r.at[pl.ds(off, C)], idxb.at[0])
            pltpu.sync_copy(arr_r.at[pl.ds(off, C), :], rows)
            pltpu.sync_copy(rows, acc.at[idxb.at[0]], add=True)
            if do_deg:
                pltpu.sync_copy(ones_v, dacc.at[idxb.at[0]], add=True)

        @pl.when(cid == 0)
        def _():
            def body(i, c):
                chunk(i, eh_r, True)
                return c
            lax.fori_loop(0, nch, body, 0)

        @pl.when(cid == 1)
        def _():
            def body(i, c):
                chunk(i, sh_r, False)
                return c
            lax.fori_loop(0, nch, body, 0)

        plsc.subcore_barrier()

        @pl.when(cid == 0)
        def _():
            _row_copy(acc, out_se, sid, n)

            @pl.when(sid == 0)
            def _():
                pltpu.sync_copy(dacc, out_dg)

        @pl.when(cid == 1)
        def _():
            _row_copy(acc, out_ss, sid, n)

    return k1


def _make_k2(n, e):
    epc = e // NC
    ept = epc // NS
    nch = ept // C

    @functools.partial(
        pl.kernel,
        out_type=[
            jax.ShapeDtypeStruct((n, 128), F32),
            jax.ShapeDtypeStruct((n, 128), F32),
        ],
        mesh=_sc_mesh(),
        scratch_types=[
            pltpu.VMEM_SHARED((n, 128), F32),
            pltpu.VMEM((1, C), jnp.int32),
            pltpu.VMEM((1, C), jnp.int32),
            pltpu.VMEM((C, 128), F32),
            pltpu.SemaphoreType.DMA,
        ],
    )
    def k2(src_r, dst_r, tab_r, z2_r, out0, out1, acc, idxs, idxd, rows, sem):
        cid = lax.axis_index("c")
        sid = lax.axis_index("s")
        _row_copy(z2_r, acc, sid, n)
        plsc.subcore_barrier()

        base = cid * epc + sid * ept

        def body(i, c):
            off = pl.multiple_of(base + i * C, 8)
            pltpu.sync_copy(src_r.at[pl.ds(off, C)], idxs.at[0])
            pltpu.sync_copy(dst_r.at[pl.ds(off, C)], idxd.at[0])
            pltpu.async_copy(tab_r.at[idxs.at[0]], rows, sem).wait()
            pltpu.sync_copy(rows, acc.at[idxd.at[0]], add=True)
            return c

        lax.fori_loop(0, nch, body, 0)
        plsc.subcore_barrier()

        @pl.when(cid == 0)
        def _():
            _row_copy(acc, out0, sid, n)

        @pl.when(cid == 1)
        def _():
            _row_copy(acc, out1, sid, n)

    return k2


def _dotT(a, b):
    return lax.dot_general(a, b, (((1,), (1,)), ((), ())),
                           preferred_element_type=F32)


def _k3_body(se_ref, ss_ref, dg_ref, h_ref, tw_ref, iw_ref, tb_ref, ib_ref,
             out_ref):
    deg = dg_ref[...]
    degc = jnp.maximum(deg, 1.0)
    h_o_r = h_ref[...] * se_ref[...] / degc
    num = _dotT(ss_ref[...], tw_ref[...]) + deg * tb_ref[...]
    h_o_s = num / degc
    out_ref[...] = (_dotT(h_o_s, iw_ref[:, :128])
                    + _dotT(h_o_r, iw_ref[:, 128:])
                    + ib_ref[...])


def _make_k3(n):
    bn = 2000
    grid = (n // bn,)
    return pl.pallas_call(
        _k3_body,
        grid=grid,
        in_specs=[
            pl.BlockSpec((bn, 128), lambda i: (i, 0)),
            pl.BlockSpec((bn, 128), lambda i: (i, 0)),
            pl.BlockSpec((bn, 1), lambda i: (i, 0)),
            pl.BlockSpec((bn, 128), lambda i: (i, 0)),
            pl.BlockSpec((128, 128), lambda i: (0, 0)),
            pl.BlockSpec((128, 256), lambda i: (0, 0)),
            pl.BlockSpec((1, 128), lambda i: (0, 0)),
            pl.BlockSpec((1, 128), lambda i: (0, 0)),
        ],
        out_specs=pl.BlockSpec((bn, 128), lambda i: (i, 0)),
        out_shape=jax.ShapeDtypeStruct((n, 128), F32),
    )


def _k4_body(eh_ref, sh_ref, rw_ref, tw_ref, rb_ref, tb_ref, out_ref,
             m2_ref, bc_ref):
    @pl.when(pl.program_id(0) == 0)
    def _():
        rw2 = rw_ref[:, 128:]
        m2_ref[...] = lax.dot_general(tw_ref[...], rw2, (((0,), (1,)), ((), ())),
                                      preferred_element_type=F32)
        bc_ref[...] = rb_ref[...] + _dotT(tb_ref[...], rw2)

    out_ref[...] = (_dotT(eh_ref[...], rw_ref[:, :128])
                    + jnp.dot(sh_ref[...], m2_ref[...],
                              preferred_element_type=F32)
                    + bc_ref[...])


def _make_k4(e):
    be = 2000
    grid = (e // be,)
    return pl.pallas_call(
        _k4_body,
        grid=grid,
        in_specs=[
            pl.BlockSpec((be, 128), lambda i: (i, 0)),
            pl.BlockSpec((be, 128), lambda i: (i, 0)),
            pl.BlockSpec((128, 256), lambda i: (0, 0)),
            pl.BlockSpec((128, 128), lambda i: (0, 0)),
            pl.BlockSpec((1, 128), lambda i: (0, 0)),
            pl.BlockSpec((1, 128), lambda i: (0, 0)),
        ],
        out_specs=pl.BlockSpec((be, 128), lambda i: (i, 0)),
        out_shape=jax.ShapeDtypeStruct((e, 128), F32),
        scratch_shapes=[
            pltpu.VMEM((128, 128), F32),
            pltpu.VMEM((1, 128), F32),
        ],
    )


def _k5_body(p0_ref, p1_ref, nm_ref, bv_ref, out_ref):
    out_ref[...] = (p0_ref[...] + p1_ref[...]) * nm_ref[...] + bv_ref[...]


def _make_k5(n):
    bn = 2000
    grid = (n // bn,)
    return pl.pallas_call(
        _k5_body,
        grid=grid,
        in_specs=[
            pl.BlockSpec((bn, 128), lambda i: (i, 0)),
            pl.BlockSpec((bn, 128), lambda i: (i, 0)),
            pl.BlockSpec((bn, 1), lambda i: (i, 0)),
            pl.BlockSpec((1, 128), lambda i: (0, 0)),
        ],
        out_specs=pl.BlockSpec((bn, 128), lambda i: (i, 0)),
        out_shape=jax.ShapeDtypeStruct((n, 128), F32),
    )


def kernel(h, norm, e_h, s_h, edge_index, text_W, text_b, inv_W, inv_b,
           rel_W, rel_b, bias_v):
    n = h.shape[0]
    e = e_h.shape[0]
    src = edge_index[0]
    dst = edge_index[1]
    z2 = jnp.zeros((n, 128), F32)
    z1 = jnp.zeros((n,), F32)
    tb = text_b.reshape(1, 128)
    ib = inv_b.reshape(1, 128)
    rb = rel_b.reshape(1, 128)
    bv = bias_v.reshape(1, 128)

    sum_e, sum_s, deg = _make_k1(n, e)(dst, e_h, s_h, z2, z1)
    h_s_r_o = _make_k3(n)(sum_e, sum_s, deg.reshape(n, 1), h, text_W,
                          inv_W, tb, ib)
    p0, p1 = _make_k2(n, e)(src, dst, h_s_r_o, z2)
    h_new = _make_k5(n)(p0, p1, norm, bv)
    e_h_new = _make_k4(e)(e_h, s_h, rel_W, text_W, rb, tb)
    return h_new, e_h_new


# speedup vs baseline: 7.7737x; 1.6050x over previous
import functools

import jax
import jax.numpy as jnp
from jax import lax
from jax.experimental import pallas as pl
from jax.experimental.pallas import tpu as pltpu
from jax.experimental.pallas import tpu_sc as plsc

F32 = jnp.float32

NC = 2
NS = 16
C = 80


def _sc_mesh():
    return plsc.VectorSubcoreMesh(core_axis_name="c", subcore_axis_name="s")


def _row_copy(src, dst, sid, n):
    per = (-(-n // NS) + 7) // 8 * 8
    last = n - (NS - 1) * per
    assert last > 0 and last % 8 == 0

    @pl.when(sid < NS - 1)
    def _():
        st = pl.multiple_of(sid * per, 8)
        pltpu.sync_copy(src.at[pl.ds(st, per), :], dst.at[pl.ds(st, per), :])

    @pl.when(sid == NS - 1)
    def _():
        st = (NS - 1) * per
        pltpu.sync_copy(src.at[pl.ds(st, last), :], dst.at[pl.ds(st, last), :])


K1_K = 1
K1_KC = K1_K * C


def _make_k1(n, e):
    ept = e // NS
    ngrp = ept // K1_KC
    npair = ngrp // 2
    assert ngrp % 2 == 0

    @functools.partial(
        pl.kernel,
        out_type=[
            jax.ShapeDtypeStruct((n, 128), F32),
            jax.ShapeDtypeStruct((n, 128), F32),
            jax.ShapeDtypeStruct((n,), F32),
        ],
        mesh=_sc_mesh(),
        scratch_types=[
            pltpu.VMEM_SHARED((n, 128), F32),
            pltpu.VMEM_SHARED((n,), F32),
            pltpu.VMEM((2, K1_K, C), jnp.int32),
            pltpu.VMEM((2, K1_KC, 128), F32),
            pltpu.VMEM((C,), F32),
            pltpu.SemaphoreType.DMA,
            pltpu.SemaphoreType.DMA,
            pltpu.SemaphoreType.DMA,
            pltpu.SemaphoreType.DMA,
        ],
    )
    def k1(dst_r, eh_r, sh_r, z2_r, z1_r, out_se, out_ss, out_dg,
           acc, dacc, idxb, rows, ones_v, isem, rsem, ssem, dsem):
        cid = lax.axis_index("c")
        sid = lax.axis_index("s")
        for t in range(C // 16):
            ones_v[pl.ds(16 * t, 16)] = jnp.full((16,), 1.0, F32)
        _row_copy(z2_r, acc, sid, n)

        @pl.when(jnp.logical_and(sid == 0, cid == 0))
        def _():
            pltpu.sync_copy(z1_r, dacc)

        plsc.subcore_barrier()

        base = sid * ept

        def start_loads(arr_r, g, b):
            off = pl.multiple_of(base + g * K1_KC, 8)
            for k in range(K1_K):
                pltpu.async_copy(dst_r.at[pl.ds(off + k * C, C)],
                                 idxb.at[b, k], isem)
            pltpu.async_copy(arr_r.at[pl.ds(off, K1_KC), :], rows.at[b], rsem)

        def wait_loads(arr_r, b):
            for k in range(K1_K):
                pltpu.make_async_copy(dst_r.at[pl.ds(0, C)],
                                      idxb.at[b, k], isem).wait()
            pltpu.make_async_copy(arr_r.at[pl.ds(0, K1_KC), :],
                                  rows.at[b], rsem).wait()

        def fire_scatters(b, do_deg):
            for k in range(K1_K):
                pltpu.async_copy(rows.at[b, pl.ds(k * C, C), :],
                                 acc.at[idxb.at[b, k]], ssem, add=True)
                if do_deg:
                    pltpu.async_copy(ones_v, dacc.at[idxb.at[b, k]], dsem,
                                     add=True)

        def drain_scatters(b, do_deg):
            for k in range(K1_K):
                pltpu.make_async_copy(rows.at[b, pl.ds(k * C, C), :],
                                      acc.at[idxb.at[b, k]], ssem).wait()
                if do_deg:
                    pltpu.make_async_copy(ones_v, dacc.at[idxb.at[b, k]],
                                          dsem).wait()

        def run(arr_r, do_deg):
            start_loads(arr_r, 0, 0)

            def pair(p, c):
                g0 = 2 * p
                wait_loads(arr_r, 0)

                @pl.when(p > 0)
                def _():
                    drain_scatters(1, do_deg)

                start_loads(arr_r, g0 + 1, 1)
                fire_scatters(0, do_deg)
                wait_loads(arr_r, 1)
                drain_scatters(0, do_deg)

                @pl.when(p < npair - 1)
                def _():
                    start_loads(arr_r, g0 + 2, 0)

                fire_scatters(1, do_deg)
                return c

            lax.fori_loop(0, npair, pair, 0)
            drain_scatters(1, do_deg)

        @pl.when(cid == 0)
        def _():
            run(eh_r, True)

        @pl.when(cid == 1)
        def _():
            run(sh_r, False)

        plsc.subcore_barrier()

        @pl.when(cid == 0)
        def _():
            _row_copy(acc, out_se, sid, n)

            @pl.when(sid == 0)
            def _():
                pltpu.sync_copy(dacc, out_dg)

        @pl.when(cid == 1)
        def _():
            _row_copy(acc, out_ss, sid, n)

    return k1


def _make_k2(n, e):
    epc = e // NC
    ept = epc // NS
    ngrp = ept // K1_KC
    npair = ngrp // 2
    tail = ngrp % 2

    @functools.partial(
        pl.kernel,
        out_type=[
            jax.ShapeDtypeStruct((n, 128), F32),
            jax.ShapeDtypeStruct((n, 128), F32),
        ],
        mesh=_sc_mesh(),
        scratch_types=[
            pltpu.VMEM_SHARED((n, 128), F32),
            pltpu.VMEM((2, K1_K, C), jnp.int32),
            pltpu.VMEM((2, K1_K, C), jnp.int32),
            pltpu.VMEM((2, K1_KC, 128), F32),
            pltpu.SemaphoreType.DMA,
            pltpu.SemaphoreType.DMA,
            pltpu.SemaphoreType.DMA,
        ],
    )
    def k2(src_r, dst_r, tab_r, z2_r, out0, out1,
           acc, idxs, idxd, rows, isem, gsem, ssem):
        cid = lax.axis_index("c")
        sid = lax.axis_index("s")
        _row_copy(z2_r, acc, sid, n)
        plsc.subcore_barrier()

        base = cid * epc + sid * ept

        def start_loads(g, b):
            off = pl.multiple_of(base + g * K1_KC, 8)
            for k in range(K1_K):
                pltpu.async_copy(src_r.at[pl.ds(off + k * C, C)],
                                 idxs.at[b, k], isem)
                pltpu.async_copy(dst_r.at[pl.ds(off + k * C, C)],
                                 idxd.at[b, k], isem)

        def wait_loads(b):
            for k in range(K1_K):
                pltpu.make_async_copy(src_r.at[pl.ds(0, C)],
                                      idxs.at[b, k], isem).wait()
                pltpu.make_async_copy(dst_r.at[pl.ds(0, C)],
                                      idxd.at[b, k], isem).wait()

        def fire_gathers(b):
            for k in range(K1_K):
                pltpu.async_copy(tab_r.at[idxs.at[b, k]],
                                 rows.at[b, pl.ds(k * C, C), :], gsem)

        def drain_gathers(b):
            for k in range(K1_K):
                pltpu.make_async_copy(tab_r.at[idxs.at[b, k]],
                                      rows.at[b, pl.ds(k * C, C), :],
                                      gsem).wait()

        def fire_scatters(b):
            for k in range(K1_K):
                pltpu.async_copy(rows.at[b, pl.ds(k * C, C), :],
                                 acc.at[idxd.at[b, k]], ssem, add=True)

        def drain_scatters(b):
            for k in range(K1_K):
                pltpu.make_async_copy(rows.at[b, pl.ds(k * C, C), :],
                                      acc.at[idxd.at[b, k]], ssem).wait()

        start_loads(0, 0)

        def pair(p, c):
            g0 = 2 * p
            wait_loads(0)
            fire_gathers(0)

            @pl.when(p > 0)
            def _():
                drain_scatters(1)

            start_loads(g0 + 1, 1)
            drain_gathers(0)
            fire_scatters(0)
            wait_loads(1)
            fire_gathers(1)
            drain_scatters(0)

            @pl.when(p < npair - 1)
            def _():
                start_loads(g0 + 2, 0)

            drain_gathers(1)
            fire_scatters(1)
            return c

        lax.fori_loop(0, npair, pair, 0)
        if tail:
            start_loads(ngrp - 1, 0)
            wait_loads(0)
            fire_gathers(0)
            drain_scatters(1)
            drain_gathers(0)
            fire_scatters(0)
            drain_scatters(0)
        else:
            drain_scatters(1)
        plsc.subcore_barrier()

        @pl.when(cid == 0)
        def _():
            _row_copy(acc, out0, sid, n)

        @pl.when(cid == 1)
        def _():
            _row_copy(acc, out1, sid, n)

    return k2


def _dotT(a, b):
    return lax.dot_general(a, b, (((1,), (1,)), ((), ())),
                           preferred_element_type=F32)


def _k3_body(se_ref, ss_ref, dg_ref, h_ref, tw_ref, iw_ref, tb_ref, ib_ref,
             out_ref):
    deg = dg_ref[...]
    degc = jnp.maximum(deg, 1.0)
    h_o_r = h_ref[...] * se_ref[...] / degc
    num = _dotT(ss_ref[...], tw_ref[...]) + deg * tb_ref[...]
    h_o_s = num / degc
    out_ref[...] = (_dotT(h_o_s, iw_ref[:, :128])
                    + _dotT(h_o_r, iw_ref[:, 128:])
                    + ib_ref[...])


def _make_k3(n):
    bn = 2000
    grid = (n // bn,)
    return pl.pallas_call(
        _k3_body,
        grid=grid,
        in_specs=[
            pl.BlockSpec((bn, 128), lambda i: (i, 0)),
            pl.BlockSpec((bn, 128), lambda i: (i, 0)),
            pl.BlockSpec((bn, 1), lambda i: (i, 0)),
            pl.BlockSpec((bn, 128), lambda i: (i, 0)),
            pl.BlockSpec((128, 128), lambda i: (0, 0)),
            pl.BlockSpec((128, 256), lambda i: (0, 0)),
            pl.BlockSpec((1, 128), lambda i: (0, 0)),
            pl.BlockSpec((1, 128), lambda i: (0, 0)),
        ],
        out_specs=pl.BlockSpec((bn, 128), lambda i: (i, 0)),
        out_shape=jax.ShapeDtypeStruct((n, 128), F32),
    )


def _k4_body(eh_ref, sh_ref, rw_ref, tw_ref, rb_ref, tb_ref, out_ref,
             m2_ref, bc_ref):
    @pl.when(pl.program_id(0) == 0)
    def _():
        rw2 = rw_ref[:, 128:]
        m2_ref[...] = lax.dot_general(tw_ref[...], rw2, (((0,), (1,)), ((), ())),
                                      preferred_element_type=F32)
        bc_ref[...] = rb_ref[...] + _dotT(tb_ref[...], rw2)

    out_ref[...] = (_dotT(eh_ref[...], rw_ref[:, :128])
                    + jnp.dot(sh_ref[...], m2_ref[...],
                              preferred_element_type=F32)
                    + bc_ref[...])


def _make_k4(e):
    be = 2000
    grid = (e // be,)
    return pl.pallas_call(
        _k4_body,
        grid=grid,
        in_specs=[
            pl.BlockSpec((be, 128), lambda i: (i, 0)),
            pl.BlockSpec((be, 128), lambda i: (i, 0)),
            pl.BlockSpec((128, 256), lambda i: (0, 0)),
            pl.BlockSpec((128, 128), lambda i: (0, 0)),
            pl.BlockSpec((1, 128), lambda i: (0, 0)),
            pl.BlockSpec((1, 128), lambda i: (0, 0)),
        ],
        out_specs=pl.BlockSpec((be, 128), lambda i: (i, 0)),
        out_shape=jax.ShapeDtypeStruct((e, 128), F32),
        scratch_shapes=[
            pltpu.VMEM((128, 128), F32),
            pltpu.VMEM((1, 128), F32),
        ],
    )


def _k5_body(p0_ref, p1_ref, nm_ref, bv_ref, out_ref):
    out_ref[...] = (p0_ref[...] + p1_ref[...]) * nm_ref[...] + bv_ref[...]


def _make_k5(n):
    bn = 2000
    grid = (n // bn,)
    return pl.pallas_call(
        _k5_body,
        grid=grid,
        in_specs=[
            pl.BlockSpec((bn, 128), lambda i: (i, 0)),
            pl.BlockSpec((bn, 128), lambda i: (i, 0)),
            pl.BlockSpec((bn, 1), lambda i: (i, 0)),
            pl.BlockSpec((1, 128), lambda i: (0, 0)),
        ],
        out_specs=pl.BlockSpec((bn, 128), lambda i: (i, 0)),
        out_shape=jax.ShapeDtypeStruct((n, 128), F32),
    )


def kernel(h, norm, e_h, s_h, edge_index, text_W, text_b, inv_W, inv_b,
           rel_W, rel_b, bias_v):
    n = h.shape[0]
    e = e_h.shape[0]
    src = edge_index[0]
    dst = edge_index[1]
    z2 = jnp.zeros((n, 128), F32)
    z1 = jnp.zeros((n,), F32)
    tb = text_b.reshape(1, 128)
    ib = inv_b.reshape(1, 128)
    rb = rel_b.reshape(1, 128)
    bv = bias_v.reshape(1, 128)

    sum_e, sum_s, deg = _make_k1(n, e)(dst, e_h, s_h, z2, z1)
    h_s_r_o = _make_k3(n)(sum_e, sum_s, deg.reshape(n, 1), h, text_W,
                          inv_W, tb, ib)
    p0, p1 = _make_k2(n, e)(src, dst, h_s_r_o, z2)
    h_new = _make_k5(n)(p0, p1, norm, bv)
    e_h_new = _make_k4(e)(e_h, s_h, rel_W, text_W, rb, tb)
    return h_new, e_h_new


# speedup vs baseline: 9.0987x; 1.1705x over previous
import functools

import jax
import jax.numpy as jnp
from jax import lax
from jax.experimental import pallas as pl
from jax.experimental.pallas import tpu as pltpu
from jax.experimental.pallas import tpu_sc as plsc

F32 = jnp.float32

NC = 2
NS = 16
C = 80


def _sc_mesh():
    return plsc.VectorSubcoreMesh(core_axis_name="c", subcore_axis_name="s")


def _row_copy(src, dst, sid, n):
    per = (-(-n // NS) + 7) // 8 * 8
    last = n - (NS - 1) * per
    assert last > 0 and last % 8 == 0

    @pl.when(sid < NS - 1)
    def _():
        st = pl.multiple_of(sid * per, 8)
        pltpu.sync_copy(src.at[pl.ds(st, per), :], dst.at[pl.ds(st, per), :])

    @pl.when(sid == NS - 1)
    def _():
        st = (NS - 1) * per
        pltpu.sync_copy(src.at[pl.ds(st, last), :], dst.at[pl.ds(st, last), :])


NB = 4


def _acc_rows(n):
    return -(-n // (NS * 64)) * (NS * 64)


def _make_k1(n, e):
    ept = e // NS
    ngrp = ept // C
    nblk = ngrp // NB
    rem = ngrp % NB
    na = _acc_rows(n)

    @functools.partial(
        pl.kernel,
        out_type=[
            jax.ShapeDtypeStruct((n, 128), F32),
            jax.ShapeDtypeStruct((n, 128), F32),
            jax.ShapeDtypeStruct((n,), F32),
        ],
        mesh=_sc_mesh(),
        scratch_types=[
            pltpu.VMEM_SHARED((na, 128), F32),
            pltpu.VMEM_SHARED((n,), F32),
            pltpu.VMEM((NB, C), jnp.int32),
            pltpu.VMEM((NB, C, 128), F32),
            pltpu.VMEM((C,), F32),
            pltpu.SemaphoreType.DMA,
            pltpu.SemaphoreType.DMA,
            pltpu.SemaphoreType.DMA,
            pltpu.SemaphoreType.DMA,
        ],
    )
    def k1(dst_r, eh_r, sh_r, z2_r, z1_r, out_se, out_ss, out_dg,
           acc, dacc, idxb, rows, ones_v, isem, rsem, ssem, dsem):
        cid = lax.axis_index("c")
        sid = lax.axis_index("s")
        for t in range(C // 16):
            ones_v[pl.ds(16 * t, 16)] = jnp.full((16,), 1.0, F32)
        _row_copy(z2_r, acc, sid, n)

        @pl.when(jnp.logical_and(sid == 0, cid == 0))
        def _():
            pltpu.sync_copy(z1_r, dacc)

        plsc.subcore_barrier()

        base = sid * ept

        def start_loads(arr_r, g, b):
            off = pl.multiple_of(base + g * C, 8)
            pltpu.async_copy(dst_r.at[pl.ds(off, C)], idxb.at[b], isem)
            pltpu.async_copy(arr_r.at[pl.ds(off, C), :], rows.at[b], rsem)

        def wait_loads(arr_r, b):
            pltpu.make_async_copy(dst_r.at[pl.ds(0, C)], idxb.at[b],
                                  isem).wait()
            pltpu.make_async_copy(arr_r.at[pl.ds(0, C), :], rows.at[b],
                                  rsem).wait()

        def fire_scatters(b, do_deg):
            pltpu.async_copy(rows.at[b], acc.at[idxb.at[b]], ssem, add=True)
            if do_deg:
                pltpu.async_copy(ones_v, dacc.at[idxb.at[b]], dsem, add=True)

        def drain_scatters(b, do_deg):
            pltpu.make_async_copy(rows.at[b], acc.at[idxb.at[b]], ssem).wait()
            if do_deg:
                pltpu.make_async_copy(ones_v, dacc.at[idxb.at[b]],
                                      dsem).wait()

        def run(arr_r, do_deg):
            for g in range(NB - 2):
                start_loads(arr_r, g, g)

            def turn(g, b):
                wait_loads(arr_r, b)

                @pl.when(g >= 2)
                def _():
                    drain_scatters((b + 2) % NB, do_deg)

                @pl.when(g + 2 < ngrp)
                def _():
                    start_loads(arr_r, g + 2, (b + 2) % NB)

                fire_scatters(b, do_deg)

            def blk(j, c):
                for b in range(NB):
                    turn(j * NB + b, b)
                return c

            lax.fori_loop(0, nblk, blk, 0)
            for r in range(rem):
                g = ngrp - rem + r
                b = g % NB
                wait_loads(arr_r, b)
                drain_scatters((b + 2) % NB, do_deg)
                fire_scatters(b, do_deg)
            drain_scatters((ngrp - 2) % NB, do_deg)
            drain_scatters((ngrp - 1) % NB, do_deg)

        @pl.when(cid == 0)
        def _():
            run(eh_r, True)

        @pl.when(cid == 1)
        def _():
            run(sh_r, False)

        plsc.subcore_barrier()

        @pl.when(cid == 0)
        def _():
            _row_copy(acc, out_se, sid, n)

            @pl.when(sid == 0)
            def _():
                pltpu.sync_copy(dacc, out_dg)

        @pl.when(cid == 1)
        def _():
            _row_copy(acc, out_ss, sid, n)

    return k1


def _make_k2(n, e):
    epc = e // NC
    ept = epc // NS
    ngrp = ept // C
    nblk = ngrp // NB
    rem = ngrp % NB
    na = _acc_rows(n)

    @functools.partial(
        pl.kernel,
        out_type=[
            jax.ShapeDtypeStruct((n, 128), F32),
            jax.ShapeDtypeStruct((n, 128), F32),
        ],
        mesh=_sc_mesh(),
        scratch_types=[
            pltpu.VMEM_SHARED((na, 128), F32),
            pltpu.VMEM((NB, C), jnp.int32),
            pltpu.VMEM((NB, C), jnp.int32),
            pltpu.VMEM((NB, C, 128), F32),
            pltpu.SemaphoreType.DMA,
            pltpu.SemaphoreType.DMA,
            pltpu.SemaphoreType.DMA,
        ],
    )
    def k2(src_r, dst_r, tab_r, z2_r, out0, out1,
           acc, idxs, idxd, rows, isem, gsem, ssem):
        cid = lax.axis_index("c")
        sid = lax.axis_index("s")
        _row_copy(z2_r, acc, sid, n)
        plsc.subcore_barrier()

        base = cid * epc + sid * ept

        def start_loads(g, b):
            off = pl.multiple_of(base + g * C, 8)
            pltpu.async_copy(src_r.at[pl.ds(off, C)], idxs.at[b], isem)
            pltpu.async_copy(dst_r.at[pl.ds(off, C)], idxd.at[b], isem)

        def wait_loads(b):
            pltpu.make_async_copy(src_r.at[pl.ds(0, C)], idxs.at[b],
                                  isem).wait()
            pltpu.make_async_copy(dst_r.at[pl.ds(0, C)], idxd.at[b],
                                  isem).wait()

        def fire_gather(b):
            pltpu.async_copy(tab_r.at[idxs.at[b]], rows.at[b], gsem)

        def drain_gather(b):
            pltpu.make_async_copy(tab_r.at[idxs.at[b]], rows.at[b],
                                  gsem).wait()

        def fire_scatter(b):
            pltpu.async_copy(rows.at[b], acc.at[idxd.at[b]], ssem, add=True)

        def drain_scatter(b):
            pltpu.make_async_copy(rows.at[b], acc.at[idxd.at[b]],
                                  ssem).wait()

        for g in range(NB - 2):
            start_loads(g, g)

        def turn(g, b):
            wait_loads(b)
            fire_gather(b)

            @pl.when(g >= 1)
            def _():
                drain_gather((b + 3) % NB)
                fire_scatter((b + 3) % NB)

            @pl.when(g >= 2)
            def _():
                drain_scatter((b + 2) % NB)

            @pl.when(g + 2 < ngrp)
            def _():
                start_loads(g + 2, (b + 2) % NB)

        def blk(j, c):
            for b in range(NB):
                turn(j * NB + b, b)
            return c

        lax.fori_loop(0, nblk, blk, 0)
        for r in range(rem):
            g = ngrp - rem + r
            b = g % NB
            wait_loads(b)
            fire_gather(b)
            drain_gather((b + 3) % NB)
            fire_scatter((b + 3) % NB)
            drain_scatter((b + 2) % NB)
        bl = (ngrp - 1) % NB
        drain_gather(bl)
        fire_scatter(bl)
        drain_scatter((bl + 3) % NB)
        drain_scatter(bl)
        plsc.subcore_barrier()

        @pl.when(cid == 0)
        def _():
            _row_copy(acc, out0, sid, n)

        @pl.when(cid == 1)
        def _():
            _row_copy(acc, out1, sid, n)

    return k2


def _dotT(a, b):
    return lax.dot_general(a, b, (((1,), (1,)), ((), ())),
                           preferred_element_type=F32)


def _k3_body(se_ref, ss_ref, dg_ref, h_ref, tw_ref, iw_ref, tb_ref, ib_ref,
             out_ref):
    deg = dg_ref[...]
    degc = jnp.maximum(deg, 1.0)
    h_o_r = h_ref[...] * se_ref[...] / degc
    num = _dotT(ss_ref[...], tw_ref[...]) + deg * tb_ref[...]
    h_o_s = num / degc
    out_ref[...] = (_dotT(h_o_s, iw_ref[:, :128])
                    + _dotT(h_o_r, iw_ref[:, 128:])
                    + ib_ref[...])


def _make_k3(n):
    bn = 2000
    grid = (n // bn,)
    return pl.pallas_call(
        _k3_body,
        grid=grid,
        in_specs=[
            pl.BlockSpec((bn, 128), lambda i: (i, 0)),
            pl.BlockSpec((bn, 128), lambda i: (i, 0)),
            pl.BlockSpec((bn, 1), lambda i: (i, 0)),
            pl.BlockSpec((bn, 128), lambda i: (i, 0)),
            pl.BlockSpec((128, 128), lambda i: (0, 0)),
            pl.BlockSpec((128, 256), lambda i: (0, 0)),
            pl.BlockSpec((1, 128), lambda i: (0, 0)),
            pl.BlockSpec((1, 128), lambda i: (0, 0)),
        ],
        out_specs=pl.BlockSpec((bn, 128), lambda i: (i, 0)),
        out_shape=jax.ShapeDtypeStruct((n, 128), F32),
    )


def _k4_body(eh_ref, sh_ref, rw_ref, tw_ref, rb_ref, tb_ref, out_ref,
             m2_ref, bc_ref):
    @pl.when(pl.program_id(0) == 0)
    def _():
        rw2 = rw_ref[:, 128:]
        m2_ref[...] = lax.dot_general(tw_ref[...], rw2, (((0,), (1,)), ((), ())),
                                      preferred_element_type=F32)
        bc_ref[...] = rb_ref[...] + _dotT(tb_ref[...], rw2)

    out_ref[...] = (_dotT(eh_ref[...], rw_ref[:, :128])
                    + jnp.dot(sh_ref[...], m2_ref[...],
                              preferred_element_type=F32)
                    + bc_ref[...])


def _make_k4(e):
    be = 2000
    grid = (e // be,)
    return pl.pallas_call(
        _k4_body,
        grid=grid,
        in_specs=[
            pl.BlockSpec((be, 128), lambda i: (i, 0)),
            pl.BlockSpec((be, 128), lambda i: (i, 0)),
            pl.BlockSpec((128, 256), lambda i: (0, 0)),
            pl.BlockSpec((128, 128), lambda i: (0, 0)),
            pl.BlockSpec((1, 128), lambda i: (0, 0)),
            pl.BlockSpec((1, 128), lambda i: (0, 0)),
        ],
        out_specs=pl.BlockSpec((be, 128), lambda i: (i, 0)),
        out_shape=jax.ShapeDtypeStruct((e, 128), F32),
        scratch_shapes=[
            pltpu.VMEM((128, 128), F32),
            pltpu.VMEM((1, 128), F32),
        ],
    )


def _k5_body(p0_ref, p1_ref, nm_ref, bv_ref, out_ref):
    out_ref[...] = (p0_ref[...] + p1_ref[...]) * nm_ref[...] + bv_ref[...]


def _make_k5(n):
    bn = 2000
    grid = (n // bn,)
    return pl.pallas_call(
        _k5_body,
        grid=grid,
        in_specs=[
            pl.BlockSpec((bn, 128), lambda i: (i, 0)),
            pl.BlockSpec((bn, 128), lambda i: (i, 0)),
            pl.BlockSpec((bn, 1), lambda i: (i, 0)),
            pl.BlockSpec((1, 128), lambda i: (0, 0)),
        ],
        out_specs=pl.BlockSpec((bn, 128), lambda i: (i, 0)),
        out_shape=jax.ShapeDtypeStruct((n, 128), F32),
    )


def kernel(h, norm, e_h, s_h, edge_index, text_W, text_b, inv_W, inv_b,
           rel_W, rel_b, bias_v):
    n = h.shape[0]
    e = e_h.shape[0]
    src = edge_index[0]
    dst = edge_index[1]
    z2 = jnp.zeros((n, 128), F32)
    z1 = jnp.zeros((n,), F32)
    tb = text_b.reshape(1, 128)
    ib = inv_b.reshape(1, 128)
    rb = rel_b.reshape(1, 128)
    bv = bias_v.reshape(1, 128)

    sum_e, sum_s, deg = _make_k1(n, e)(dst, e_h, s_h, z2, z1)
    h_s_r_o = _make_k3(n)(sum_e, sum_s, deg.reshape(n, 1), h, text_W,
                          inv_W, tb, ib)
    p0, p1 = _make_k2(n, e)(src, dst, h_s_r_o, z2)
    h_new = _make_k5(n)(p0, p1, norm, bv)
    e_h_new = _make_k4(e)(e_h, s_h, rel_W, text_W, rb, tb)
    return h_new, e_h_new


# speedup vs baseline: 9.6838x; 1.0643x over previous
import functools

import jax
import jax.numpy as jnp
from jax import lax
from jax.experimental import pallas as pl
from jax.experimental.pallas import tpu as pltpu
from jax.experimental.pallas import tpu_sc as plsc

F32 = jnp.float32

NC = 2
NS = 16
C = 80


def _sc_mesh():
    return plsc.VectorSubcoreMesh(core_axis_name="c", subcore_axis_name="s")


def _row_copy(src, dst, sid, n):
    per = (-(-n // NS) + 7) // 8 * 8
    last = n - (NS - 1) * per
    assert last > 0 and last % 8 == 0

    @pl.when(sid < NS - 1)
    def _():
        st = pl.multiple_of(sid * per, 8)
        pltpu.sync_copy(src.at[pl.ds(st, per), :], dst.at[pl.ds(st, per), :])

    @pl.when(sid == NS - 1)
    def _():
        st = (NS - 1) * per
        pltpu.sync_copy(src.at[pl.ds(st, last), :], dst.at[pl.ds(st, last), :])


NB = 4


def _acc_rows(n):
    return -(-n // (NS * 64)) * (NS * 64)


def _make_k1(n, e):
    ept = e // NS
    ngrp = ept // C
    nblk = ngrp // NB
    rem = ngrp % NB
    na = _acc_rows(n)

    @functools.partial(
        pl.kernel,
        out_type=[
            jax.ShapeDtypeStruct((n, 128), F32),
            jax.ShapeDtypeStruct((n, 128), F32),
            jax.ShapeDtypeStruct((n,), F32),
        ],
        mesh=_sc_mesh(),
        scratch_types=[
            pltpu.VMEM_SHARED((na, 128), F32),
            pltpu.VMEM_SHARED((n,), F32),
            pltpu.VMEM((NB, C), jnp.int32),
            pltpu.VMEM((NB, C, 128), F32),
            pltpu.VMEM((C,), F32),
            pltpu.SemaphoreType.DMA,
            pltpu.SemaphoreType.DMA,
            pltpu.SemaphoreType.DMA,
            pltpu.SemaphoreType.DMA,
        ],
    )
    def k1(dst_r, eh_r, sh_r, z2_r, z1_r, out_se, out_ss, out_dg,
           acc, dacc, idxb, rows, ones_v, isem, rsem, ssem, dsem):
        cid = lax.axis_index("c")
        sid = lax.axis_index("s")
        for t in range(C // 16):
            ones_v[pl.ds(16 * t, 16)] = jnp.full((16,), 1.0, F32)
        _row_copy(z2_r, acc, sid, n)

        @pl.when(jnp.logical_and(sid == 0, cid == 0))
        def _():
            pltpu.sync_copy(z1_r, dacc)

        plsc.subcore_barrier()

        base = sid * ept

        def start_loads(arr_r, g, b):
            off = pl.multiple_of(base + g * C, 8)
            pltpu.async_copy(dst_r.at[pl.ds(off, C)], idxb.at[b], isem)
            pltpu.async_copy(arr_r.at[pl.ds(off, C), :], rows.at[b], rsem)

        def wait_loads(arr_r, b):
            pltpu.make_async_copy(dst_r.at[pl.ds(0, C)], idxb.at[b],
                                  isem).wait()
            pltpu.make_async_copy(arr_r.at[pl.ds(0, C), :], rows.at[b],
                                  rsem).wait()

        def fire_scatters(b, do_deg):
            pltpu.async_copy(rows.at[b], acc.at[idxb.at[b]], ssem, add=True)
            if do_deg:
                pltpu.async_copy(ones_v, dacc.at[idxb.at[b]], dsem, add=True)

        def drain_scatters(b, do_deg):
            pltpu.make_async_copy(rows.at[b], acc.at[idxb.at[b]], ssem).wait()
            if do_deg:
                pltpu.make_async_copy(ones_v, dacc.at[idxb.at[b]],
                                      dsem).wait()

        def run(arr_r, do_deg):
            for g in range(NB - 2):
                start_loads(arr_r, g, g)

            def turn(g, b):
                wait_loads(arr_r, b)

                @pl.when(g >= 2)
                def _():
                    drain_scatters((b + 2) % NB, do_deg)

                @pl.when(g + 2 < ngrp)
                def _():
                    start_loads(arr_r, g + 2, (b + 2) % NB)

                fire_scatters(b, do_deg)

            def blk(j, c):
                for b in range(NB):
                    turn(j * NB + b, b)
                return c

            lax.fori_loop(0, nblk, blk, 0)
            for r in range(rem):
                g = ngrp - rem + r
                b = g % NB
                wait_loads(arr_r, b)
                drain_scatters((b + 2) % NB, do_deg)
                fire_scatters(b, do_deg)
            drain_scatters((ngrp - 2) % NB, do_deg)
            drain_scatters((ngrp - 1) % NB, do_deg)

        @pl.when(cid == 0)
        def _():
            run(eh_r, True)

        @pl.when(cid == 1)
        def _():
            run(sh_r, False)

        plsc.subcore_barrier()

        @pl.when(cid == 0)
        def _():
            _row_copy(acc, out_se, sid, n)

            @pl.when(sid == 0)
            def _():
                pltpu.sync_copy(dacc, out_dg)

        @pl.when(cid == 1)
        def _():
            _row_copy(acc, out_ss, sid, n)

    return k1


def _make_k2(n, e):
    epc = e // NC
    ept = epc // NS
    ngrp = ept // C
    nblk = ngrp // NB
    rem = ngrp % NB
    na = _acc_rows(n)

    @functools.partial(
        pl.kernel,
        out_type=[
            jax.ShapeDtypeStruct((n, 128), F32),
            jax.ShapeDtypeStruct((n, 128), F32),
        ],
        mesh=_sc_mesh(),
        scratch_types=[
            pltpu.VMEM_SHARED((na, 128), F32),
            pltpu.VMEM((NB, C), jnp.int32),
            pltpu.VMEM((NB, C), jnp.int32),
            pltpu.VMEM((NB, C, 128), F32),
            pltpu.SemaphoreType.DMA,
            pltpu.SemaphoreType.DMA,
            pltpu.SemaphoreType.DMA,
        ],
    )
    def k2(src_r, dst_r, tab_r, z2_r, out0, out1,
           acc, idxs, idxd, rows, isem, gsem, ssem):
        cid = lax.axis_index("c")
        sid = lax.axis_index("s")
        _row_copy(z2_r, acc, sid, n)
        plsc.subcore_barrier()

        base = cid * epc + sid * ept

        def start_loads(g, b):
            off = pl.multiple_of(base + g * C, 8)
            pltpu.async_copy(src_r.at[pl.ds(off, C)], idxs.at[b], isem)
            pltpu.async_copy(dst_r.at[pl.ds(off, C)], idxd.at[b], isem)

        def wait_loads(b):
            pltpu.make_async_copy(src_r.at[pl.ds(0, C)], idxs.at[b],
                                  isem).wait()
            pltpu.make_async_copy(dst_r.at[pl.ds(0, C)], idxd.at[b],
                                  isem).wait()

        def fire_gather(b):
            pltpu.async_copy(tab_r.at[idxs.at[b]], rows.at[b], gsem)

        def drain_gather(b):
            pltpu.make_async_copy(tab_r.at[idxs.at[b]], rows.at[b],
                                  gsem).wait()

        def fire_scatter(b):
            pltpu.async_copy(rows.at[b], acc.at[idxd.at[b]], ssem, add=True)

        def drain_scatter(b):
            pltpu.make_async_copy(rows.at[b], acc.at[idxd.at[b]],
                                  ssem).wait()

        for g in range(NB - 2):
            start_loads(g, g)

        def turn(g, b):
            wait_loads(b)
            fire_gather(b)

            @pl.when(g >= 1)
            def _():
                drain_gather((b + 3) % NB)
                fire_scatter((b + 3) % NB)

            @pl.when(g >= 2)
            def _():
                drain_scatter((b + 2) % NB)

            @pl.when(g + 2 < ngrp)
            def _():
                start_loads(g + 2, (b + 2) % NB)

        def blk(j, c):
            for b in range(NB):
                turn(j * NB + b, b)
            return c

        lax.fori_loop(0, nblk, blk, 0)
        for r in range(rem):
            g = ngrp - rem + r
            b = g % NB
            wait_loads(b)
            fire_gather(b)
            drain_gather((b + 3) % NB)
            fire_scatter((b + 3) % NB)
            drain_scatter((b + 2) % NB)
        bl = (ngrp - 1) % NB
        drain_gather(bl)
        fire_scatter(bl)
        drain_scatter((bl + 3) % NB)
        drain_scatter(bl)
        plsc.subcore_barrier()

        @pl.when(cid == 0)
        def _():
            _row_copy(acc, out0, sid, n)

        @pl.when(cid == 1)
        def _():
            _row_copy(acc, out1, sid, n)

    return k2


def _dotT(a, b):
    return lax.dot_general(a, b, (((1,), (1,)), ((), ())),
                           preferred_element_type=F32)


def _k3_body(se_ref, ss_ref, dg_ref, h_ref, tw_ref, iw_ref, tb_ref, ib_ref,
             out_ref):
    deg = dg_ref[...]
    degc = jnp.maximum(deg, 1.0)
    h_o_r = h_ref[...] * se_ref[...] / degc
    num = _dotT(ss_ref[...], tw_ref[...]) + deg * tb_ref[...]
    h_o_s = num / degc
    out_ref[...] = (_dotT(h_o_s, iw_ref[:, :128])
                    + _dotT(h_o_r, iw_ref[:, 128:])
                    + ib_ref[...])


def _make_k3(n):
    bn = 2000
    grid = (n // bn,)
    return pl.pallas_call(
        _k3_body,
        grid=grid,
        in_specs=[
            pl.BlockSpec((bn, 128), lambda i: (i, 0)),
            pl.BlockSpec((bn, 128), lambda i: (i, 0)),
            pl.BlockSpec((bn, 1), lambda i: (i, 0)),
            pl.BlockSpec((bn, 128), lambda i: (i, 0)),
            pl.BlockSpec((128, 128), lambda i: (0, 0)),
            pl.BlockSpec((128, 256), lambda i: (0, 0)),
            pl.BlockSpec((1, 128), lambda i: (0, 0)),
            pl.BlockSpec((1, 128), lambda i: (0, 0)),
        ],
        out_specs=pl.BlockSpec((bn, 128), lambda i: (i, 0)),
        out_shape=jax.ShapeDtypeStruct((n, 128), F32),
    )


def _k4_body(eh_ref, sh_ref, rw_ref, tw_ref, rb_ref, tb_ref, out_ref,
             wc_ref, bc_ref):
    @pl.when(pl.program_id(0) == 0)
    def _():
        rw2 = rw_ref[:, 128:]
        wc_ref[:128, :] = jnp.transpose(rw_ref[:, :128])
        wc_ref[128:, :] = lax.dot_general(tw_ref[...], rw2,
                                          (((0,), (1,)), ((), ())),
                                          preferred_element_type=F32)
        bc_ref[...] = rb_ref[...] + _dotT(tb_ref[...], rw2)

    x = jnp.concatenate([eh_ref[...], sh_ref[...]], axis=1)
    out_ref[...] = (jnp.dot(x, wc_ref[...], preferred_element_type=F32)
                    + bc_ref[...])


def _make_k4(e):
    be = 4000
    grid = (e // be,)
    return pl.pallas_call(
        _k4_body,
        grid=grid,
        in_specs=[
            pl.BlockSpec((be, 128), lambda i: (i, 0)),
            pl.BlockSpec((be, 128), lambda i: (i, 0)),
            pl.BlockSpec((128, 256), lambda i: (0, 0)),
            pl.BlockSpec((128, 128), lambda i: (0, 0)),
            pl.BlockSpec((1, 128), lambda i: (0, 0)),
            pl.BlockSpec((1, 128), lambda i: (0, 0)),
        ],
        out_specs=pl.BlockSpec((be, 128), lambda i: (i, 0)),
        out_shape=jax.ShapeDtypeStruct((e, 128), F32),
        scratch_shapes=[
            pltpu.VMEM((256, 128), F32),
            pltpu.VMEM((1, 128), F32),
        ],
    )


def _k5_body(p0_ref, p1_ref, nm_ref, bv_ref, out_ref):
    out_ref[...] = (p0_ref[...] + p1_ref[...]) * nm_ref[...] + bv_ref[...]


def _make_k5(n):
    bn = 2000
    grid = (n // bn,)
    return pl.pallas_call(
        _k5_body,
        grid=grid,
        in_specs=[
            pl.BlockSpec((bn, 128), lambda i: (i, 0)),
            pl.BlockSpec((bn, 128), lambda i: (i, 0)),
            pl.BlockSpec((bn, 1), lambda i: (i, 0)),
            pl.BlockSpec((1, 128), lambda i: (0, 0)),
        ],
        out_specs=pl.BlockSpec((bn, 128), lambda i: (i, 0)),
        out_shape=jax.ShapeDtypeStruct((n, 128), F32),
    )


def kernel(h, norm, e_h, s_h, edge_index, text_W, text_b, inv_W, inv_b,
           rel_W, rel_b, bias_v):
    n = h.shape[0]
    e = e_h.shape[0]
    src = edge_index[0]
    dst = edge_index[1]
    z2 = jnp.zeros((n, 128), F32)
    z1 = jnp.zeros((n,), F32)
    tb = text_b.reshape(1, 128)
    ib = inv_b.reshape(1, 128)
    rb = rel_b.reshape(1, 128)
    bv = bias_v.reshape(1, 128)

    sum_e, sum_s, deg = _make_k1(n, e)(dst, e_h, s_h, z2, z1)
    h_s_r_o = _make_k3(n)(sum_e, sum_s, deg.reshape(n, 1), h, text_W,
                          inv_W, tb, ib)
    p0, p1 = _make_k2(n, e)(src, dst, h_s_r_o, z2)
    h_new = _make_k5(n)(p0, p1, norm, bv)
    e_h_new = _make_k4(e)(e_h, s_h, rel_W, text_W, rb, tb)
    return h_new, e_h_new


# speedup vs baseline: 9.7693x; 1.0088x over previous
import functools

import jax
import jax.numpy as jnp
from jax import lax
from jax.experimental import pallas as pl
from jax.experimental.pallas import tpu as pltpu
from jax.experimental.pallas import tpu_sc as plsc

F32 = jnp.float32

NC = 2
NS = 16
C = 80


def _sc_mesh():
    return plsc.VectorSubcoreMesh(core_axis_name="c", subcore_axis_name="s")


def _row_copy(src, dst, sid, n):
    per = (-(-n // NS) + 7) // 8 * 8
    last = n - (NS - 1) * per
    assert last > 0 and last % 8 == 0

    @pl.when(sid < NS - 1)
    def _():
        st = pl.multiple_of(sid * per, 8)
        pltpu.sync_copy(src.at[pl.ds(st, per), :], dst.at[pl.ds(st, per), :])

    @pl.when(sid == NS - 1)
    def _():
        st = (NS - 1) * per
        pltpu.sync_copy(src.at[pl.ds(st, last), :], dst.at[pl.ds(st, last), :])


def _zero_acc(zb, acc, sid, n):
    zh = 40

    def zrow(i, c):
        for t in range(8):
            zb[i, pl.ds(16 * t, 16)] = jnp.zeros((16,), F32)
        return c

    lax.fori_loop(0, zh, zrow, 0)
    per = (-(-n // NS) + zh - 1) // zh * zh
    last = n - (NS - 1) * per
    assert last > 0 and last % zh == 0

    @pl.when(sid < NS - 1)
    def _():
        st = pl.multiple_of(sid * per, 8)
        for q in range(per // zh):
            pltpu.sync_copy(zb, acc.at[pl.ds(st + q * zh, zh), :])

    @pl.when(sid == NS - 1)
    def _():
        st = (NS - 1) * per
        for q in range(last // zh):
            pltpu.sync_copy(zb, acc.at[pl.ds(st + q * zh, zh), :])


NB = 4


def _acc_rows(n):
    return -(-n // (NS * 64)) * (NS * 64)


def _make_k1(n, e):
    ept = e // NS
    ngrp = ept // C
    nblk = ngrp // NB
    rem = ngrp % NB
    na = _acc_rows(n)

    @functools.partial(
        pl.kernel,
        out_type=[
            jax.ShapeDtypeStruct((n, 128), F32),
            jax.ShapeDtypeStruct((n, 128), F32),
            jax.ShapeDtypeStruct((n,), F32),
        ],
        mesh=_sc_mesh(),
        scratch_types=[
            pltpu.VMEM_SHARED((na, 128), F32),
            pltpu.VMEM_SHARED((n,), F32),
            pltpu.VMEM((NB, C), jnp.int32),
            pltpu.VMEM((NB, C, 128), F32),
            pltpu.VMEM((C,), F32),
            pltpu.SemaphoreType.DMA,
            pltpu.SemaphoreType.DMA,
            pltpu.SemaphoreType.DMA,
            pltpu.SemaphoreType.DMA,
        ],
    )
    def k1(dst_r, eh_r, sh_r, z2_r, z1_r, out_se, out_ss, out_dg,
           acc, dacc, idxb, rows, ones_v, isem, rsem, ssem, dsem):
        cid = lax.axis_index("c")
        sid = lax.axis_index("s")
        for t in range(C // 16):
            ones_v[pl.ds(16 * t, 16)] = jnp.full((16,), 1.0, F32)
        _row_copy(z2_r, acc, sid, n)

        @pl.when(jnp.logical_and(sid == 0, cid == 0))
        def _():
            pltpu.sync_copy(z1_r, dacc)

        plsc.subcore_barrier()

        base = sid * ept

        def start_loads(arr_r, g, b):
            off = pl.multiple_of(base + g * C, 8)
            pltpu.async_copy(dst_r.at[pl.ds(off, C)], idxb.at[b], isem)
            pltpu.async_copy(arr_r.at[pl.ds(off, C), :], rows.at[b], rsem)

        def wait_loads(arr_r, b):
            pltpu.make_async_copy(dst_r.at[pl.ds(0, C)], idxb.at[b],
                                  isem).wait()
            pltpu.make_async_copy(arr_r.at[pl.ds(0, C), :], rows.at[b],
                                  rsem).wait()

        def fire_scatters(b, do_deg):
            pltpu.async_copy(rows.at[b], acc.at[idxb.at[b]], ssem, add=True)
            if do_deg:
                pltpu.async_copy(ones_v, dacc.at[idxb.at[b]], dsem, add=True)

        def drain_scatters(b, do_deg):
            pltpu.make_async_copy(rows.at[b], acc.at[idxb.at[b]], ssem).wait()
            if do_deg:
                pltpu.make_async_copy(ones_v, dacc.at[idxb.at[b]],
                                      dsem).wait()

        def run(arr_r, do_deg):
            for g in range(NB - 2):
                start_loads(arr_r, g, g)

            def turn(g, b):
                wait_loads(arr_r, b)

                @pl.when(g >= 2)
                def _():
                    drain_scatters((b + 2) % NB, do_deg)

                @pl.when(g + 2 < ngrp)
                def _():
                    start_loads(arr_r, g + 2, (b + 2) % NB)

                fire_scatters(b, do_deg)

            def blk(j, c):
                for b in range(NB):
                    turn(j * NB + b, b)
                return c

            lax.fori_loop(0, nblk, blk, 0)
            for r in range(rem):
                g = ngrp - rem + r
                b = g % NB
                wait_loads(arr_r, b)
                drain_scatters((b + 2) % NB, do_deg)
                fire_scatters(b, do_deg)
            drain_scatters((ngrp - 2) % NB, do_deg)
            drain_scatters((ngrp - 1) % NB, do_deg)

        @pl.when(cid == 0)
        def _():
            run(eh_r, True)

        @pl.when(cid == 1)
        def _():
            run(sh_r, False)

        plsc.subcore_barrier()

        @pl.when(cid == 0)
        def _():
            _row_copy(acc, out_se, sid, n)

            @pl.when(sid == 0)
            def _():
                pltpu.sync_copy(dacc, out_dg)

        @pl.when(cid == 1)
        def _():
            _row_copy(acc, out_ss, sid, n)

    return k1


def _make_k2(n, e):
    epc = e // NC
    ept = epc // NS
    ngrp = ept // C
    nblk = ngrp // NB
    rem = ngrp % NB
    na = _acc_rows(n)

    @functools.partial(
        pl.kernel,
        out_type=[
            jax.ShapeDtypeStruct((2, n, 128), F32),
        ],
        mesh=_sc_mesh(),
        scratch_types=[
            pltpu.VMEM_SHARED((na, 128), F32),
            pltpu.VMEM((NB, C), jnp.int32),
            pltpu.VMEM((NB, C), jnp.int32),
            pltpu.VMEM((NB, C, 128), F32),
            pltpu.VMEM((40, 128), F32),
            pltpu.SemaphoreType.DMA,
            pltpu.SemaphoreType.DMA,
            pltpu.SemaphoreType.DMA,
        ],
    )
    def k2(src_r, dst_r, tab_r, outp,
           acc, idxs, idxd, rows, zb, isem, gsem, ssem):
        cid = lax.axis_index("c")
        sid = lax.axis_index("s")
        _zero_acc(zb, acc, sid, n)
        plsc.subcore_barrier()

        base = cid * epc + sid * ept

        def start_loads(g, b):
            off = pl.multiple_of(base + g * C, 8)
            pltpu.async_copy(src_r.at[pl.ds(off, C)], idxs.at[b], isem)
            pltpu.async_copy(dst_r.at[pl.ds(off, C)], idxd.at[b], isem)

        def wait_loads(b):
            pltpu.make_async_copy(src_r.at[pl.ds(0, C)], idxs.at[b],
                                  isem).wait()
            pltpu.make_async_copy(dst_r.at[pl.ds(0, C)], idxd.at[b],
                                  isem).wait()

        def fire_gather(b):
            pltpu.async_copy(tab_r.at[idxs.at[b]], rows.at[b], gsem)

        def drain_gather(b):
            pltpu.make_async_copy(tab_r.at[idxs.at[b]], rows.at[b],
                                  gsem).wait()

        def fire_scatter(b):
            pltpu.async_copy(rows.at[b], acc.at[idxd.at[b]], ssem, add=True)

        def drain_scatter(b):
            pltpu.make_async_copy(rows.at[b], acc.at[idxd.at[b]],
                                  ssem).wait()

        for g in range(NB - 2):
            start_loads(g, g)

        def turn(g, b):
            wait_loads(b)
            fire_gather(b)

            @pl.when(g >= 1)
            def _():
                drain_gather((b + 3) % NB)
                fire_scatter((b + 3) % NB)

            @pl.when(g >= 2)
            def _():
                drain_scatter((b + 2) % NB)

            @pl.when(g + 2 < ngrp)
            def _():
                start_loads(g + 2, (b + 2) % NB)

        def blk(j, c):
            for b in range(NB):
                turn(j * NB + b, b)
            return c

        lax.fori_loop(0, nblk, blk, 0)
        for r in range(rem):
            g = ngrp - rem + r
            b = g % NB
            wait_loads(b)
            fire_gather(b)
            drain_gather((b + 3) % NB)
            fire_scatter((b + 3) % NB)
            drain_scatter((b + 2) % NB)
        bl = (ngrp - 1) % NB
        drain_gather(bl)
        fire_scatter(bl)
        drain_scatter((bl + 3) % NB)
        drain_scatter(bl)
        plsc.subcore_barrier()
        _row_copy(acc, outp.at[cid], sid, n)

    return k2


def _dotT(a, b):
    return lax.dot_general(a, b, (((1,), (1,)), ((), ())),
                           preferred_element_type=F32)


def _k3_body(se_ref, ss_ref, dg_ref, h_ref, tw_ref, iw_ref, tb_ref, ib_ref,
             out_ref):
    deg = dg_ref[...]
    degc = jnp.maximum(deg, 1.0)
    h_o_r = h_ref[...] * se_ref[...] / degc
    num = _dotT(ss_ref[...], tw_ref[...]) + deg * tb_ref[...]
    h_o_s = num / degc
    out_ref[...] = (_dotT(h_o_s, iw_ref[:, :128])
                    + _dotT(h_o_r, iw_ref[:, 128:])
                    + ib_ref[...])


def _make_k3(n):
    bn = 2000
    grid = (n // bn,)
    return pl.pallas_call(
        _k3_body,
        grid=grid,
        in_specs=[
            pl.BlockSpec((bn, 128), lambda i: (i, 0)),
            pl.BlockSpec((bn, 128), lambda i: (i, 0)),
            pl.BlockSpec((bn, 1), lambda i: (i, 0)),
            pl.BlockSpec((bn, 128), lambda i: (i, 0)),
            pl.BlockSpec((128, 128), lambda i: (0, 0)),
            pl.BlockSpec((128, 256), lambda i: (0, 0)),
            pl.BlockSpec((1, 128), lambda i: (0, 0)),
            pl.BlockSpec((1, 128), lambda i: (0, 0)),
        ],
        out_specs=pl.BlockSpec((bn, 128), lambda i: (i, 0)),
        out_shape=jax.ShapeDtypeStruct((n, 128), F32),
    )


def _k4_body(eh_ref, sh_ref, rw_ref, tw_ref, rb_ref, tb_ref, out_ref,
             wc_ref, bc_ref):
    @pl.when(pl.program_id(0) == 0)
    def _():
        rw2 = rw_ref[:, 128:]
        wc_ref[:128, :] = jnp.transpose(rw_ref[:, :128])
        wc_ref[128:, :] = lax.dot_general(tw_ref[...], rw2,
                                          (((0,), (1,)), ((), ())),
                                          preferred_element_type=F32)
        bc_ref[...] = rb_ref[...] + _dotT(tb_ref[...], rw2)

    x = jnp.concatenate([eh_ref[...], sh_ref[...]], axis=1)
    out_ref[...] = (jnp.dot(x, wc_ref[...], preferred_element_type=F32)
                    + bc_ref[...])


def _make_k4(e):
    be = 4000
    grid = (e // be,)
    return pl.pallas_call(
        _k4_body,
        grid=grid,
        in_specs=[
            pl.BlockSpec((be, 128), lambda i: (i, 0)),
            pl.BlockSpec((be, 128), lambda i: (i, 0)),
            pl.BlockSpec((128, 256), lambda i: (0, 0)),
            pl.BlockSpec((128, 128), lambda i: (0, 0)),
            pl.BlockSpec((1, 128), lambda i: (0, 0)),
            pl.BlockSpec((1, 128), lambda i: (0, 0)),
        ],
        out_specs=pl.BlockSpec((be, 128), lambda i: (i, 0)),
        out_shape=jax.ShapeDtypeStruct((e, 128), F32),
        scratch_shapes=[
            pltpu.VMEM((256, 128), F32),
            pltpu.VMEM((1, 128), F32),
        ],
    )


def _k5_body(pp_ref, nm_ref, bv_ref, out_ref):
    out_ref[...] = (pp_ref[0] + pp_ref[1]) * nm_ref[...] + bv_ref[...]


def _make_k5(n):
    bn = 2000
    grid = (n // bn,)
    return pl.pallas_call(
        _k5_body,
        grid=grid,
        in_specs=[
            pl.BlockSpec((2, bn, 128), lambda i: (0, i, 0)),
            pl.BlockSpec((bn, 1), lambda i: (i, 0)),
            pl.BlockSpec((1, 128), lambda i: (0, 0)),
        ],
        out_specs=pl.BlockSpec((bn, 128), lambda i: (i, 0)),
        out_shape=jax.ShapeDtypeStruct((n, 128), F32),
    )


def kernel(h, norm, e_h, s_h, edge_index, text_W, text_b, inv_W, inv_b,
           rel_W, rel_b, bias_v):
    n = h.shape[0]
    e = e_h.shape[0]
    src = edge_index[0]
    dst = edge_index[1]
    z2 = jnp.zeros((n, 128), F32)
    z1 = jnp.zeros((n,), F32)
    tb = text_b.reshape(1, 128)
    ib = inv_b.reshape(1, 128)
    rb = rel_b.reshape(1, 128)
    bv = bias_v.reshape(1, 128)

    sum_e, sum_s, deg = _make_k1(n, e)(dst, e_h, s_h, z2, z1)
    h_s_r_o = _make_k3(n)(sum_e, sum_s, deg.reshape(n, 1), h, text_W,
                          inv_W, tb, ib)
    (presum,) = _make_k2(n, e)(src, dst, h_s_r_o)
    h_new = _make_k5(n)(presum, norm, bv)
    e_h_new = _make_k4(e)(e_h, s_h, rel_W, text_W, rb, tb)
    return h_new, e_h_new


# speedup vs baseline: 9.8689x; 1.0102x over previous
import functools

import jax
import jax.numpy as jnp
from jax import lax
from jax.experimental import pallas as pl
from jax.experimental.pallas import tpu as pltpu
from jax.experimental.pallas import tpu_sc as plsc

F32 = jnp.float32

NC = 2
NS = 16
C = 80


def _sc_mesh():
    return plsc.VectorSubcoreMesh(core_axis_name="c", subcore_axis_name="s")


def _row_copy(src, dst, sid, n):
    per = (-(-n // NS) + 7) // 8 * 8
    last = n - (NS - 1) * per
    assert last > 0 and last % 8 == 0

    @pl.when(sid < NS - 1)
    def _():
        st = pl.multiple_of(sid * per, 8)
        pltpu.sync_copy(src.at[pl.ds(st, per), :], dst.at[pl.ds(st, per), :])

    @pl.when(sid == NS - 1)
    def _():
        st = (NS - 1) * per
        pltpu.sync_copy(src.at[pl.ds(st, last), :], dst.at[pl.ds(st, last), :])


def _zero_acc(zb, acc, sid, n):
    zh = 40

    def zrow(i, c):
        for t in range(8):
            zb[i, pl.ds(16 * t, 16)] = jnp.zeros((16,), F32)
        return c

    lax.fori_loop(0, zh, zrow, 0)
    per = (-(-n // NS) + zh - 1) // zh * zh
    last = n - (NS - 1) * per
    assert last > 0 and last % zh == 0

    @pl.when(sid < NS - 1)
    def _():
        st = pl.multiple_of(sid * per, 8)
        for q in range(per // zh):
            pltpu.sync_copy(zb, acc.at[pl.ds(st + q * zh, zh), :])

    @pl.when(sid == NS - 1)
    def _():
        st = (NS - 1) * per
        for q in range(last // zh):
            pltpu.sync_copy(zb, acc.at[pl.ds(st + q * zh, zh), :])


NB = 4


def _acc_rows(n):
    return -(-n // (NS * 64)) * (NS * 64)


def _make_k1(n, e):
    ept = e // NS
    ngrp = ept // C
    nblk = ngrp // NB
    rem = ngrp % NB
    na = _acc_rows(n)

    @functools.partial(
        pl.kernel,
        out_type=[
            jax.ShapeDtypeStruct((n, 128), F32),
            jax.ShapeDtypeStruct((n, 128), F32),
            jax.ShapeDtypeStruct((n,), F32),
        ],
        mesh=_sc_mesh(),
        scratch_types=[
            pltpu.VMEM_SHARED((na, 128), F32),
            pltpu.VMEM_SHARED((n,), F32),
            pltpu.VMEM((NB, C), jnp.int32),
            pltpu.VMEM((NB, C, 128), F32),
            pltpu.VMEM((C,), F32),
            pltpu.VMEM((40, 128), F32),
            pltpu.VMEM((640,), F32),
            pltpu.SemaphoreType.DMA,
            pltpu.SemaphoreType.DMA,
            pltpu.SemaphoreType.DMA,
            pltpu.SemaphoreType.DMA,
        ],
    )
    def k1(dst_r, eh_r, sh_r, out_se, out_ss, out_dg,
           acc, dacc, idxb, rows, ones_v, zb, zv, isem, rsem, ssem, dsem):
        cid = lax.axis_index("c")
        sid = lax.axis_index("s")
        for t in range(C // 16):
            ones_v[pl.ds(16 * t, 16)] = jnp.full((16,), 1.0, F32)
        _zero_acc(zb, acc, sid, n)

        def zvrow(i, c):
            zv[pl.ds(16 * i, 16)] = jnp.zeros((16,), F32)
            return c

        lax.fori_loop(0, 40, zvrow, 0)
        dper = 640
        dlast = n - (NS - 1) * dper
        assert 0 < dlast <= dper and dlast % 8 == 0

        @pl.when(sid < NS - 1)
        def _():
            st = pl.multiple_of(sid * dper, 8)
            pltpu.sync_copy(zv, dacc.at[pl.ds(st, dper)])

        @pl.when(sid == NS - 1)
        def _():
            pltpu.sync_copy(zv.at[pl.ds(0, dlast)],
                            dacc.at[pl.ds((NS - 1) * dper, dlast)])

        plsc.subcore_barrier()

        base = sid * ept

        def start_loads(arr_r, g, b):
            off = pl.multiple_of(base + g * C, 8)
            pltpu.async_copy(dst_r.at[pl.ds(off, C)], idxb.at[b], isem)
            pltpu.async_copy(arr_r.at[pl.ds(off, C), :], rows.at[b], rsem)

        def wait_loads(arr_r, b):
            pltpu.make_async_copy(dst_r.at[pl.ds(0, C)], idxb.at[b],
                                  isem).wait()
            pltpu.make_async_copy(arr_r.at[pl.ds(0, C), :], rows.at[b],
                                  rsem).wait()

        def fire_scatters(b, do_deg):
            pltpu.async_copy(rows.at[b], acc.at[idxb.at[b]], ssem, add=True)
            if do_deg:
                pltpu.async_copy(ones_v, dacc.at[idxb.at[b]], dsem, add=True)

        def drain_scatters(b, do_deg):
            pltpu.make_async_copy(rows.at[b], acc.at[idxb.at[b]], ssem).wait()
            if do_deg:
                pltpu.make_async_copy(ones_v, dacc.at[idxb.at[b]],
                                      dsem).wait()

        def run(arr_r, do_deg):
            for g in range(NB - 2):
                start_loads(arr_r, g, g)

            def turn(g, b):
                wait_loads(arr_r, b)

                @pl.when(g >= 2)
                def _():
                    drain_scatters((b + 2) % NB, do_deg)

                @pl.when(g + 2 < ngrp)
                def _():
                    start_loads(arr_r, g + 2, (b + 2) % NB)

                fire_scatters(b, do_deg)

            def blk(j, c):
                for b in range(NB):
                    turn(j * NB + b, b)
                return c

            lax.fori_loop(0, nblk, blk, 0)
            for r in range(rem):
                g = ngrp - rem + r
                b = g % NB
                wait_loads(arr_r, b)
                drain_scatters((b + 2) % NB, do_deg)
                fire_scatters(b, do_deg)
            drain_scatters((ngrp - 2) % NB, do_deg)
            drain_scatters((ngrp - 1) % NB, do_deg)

        @pl.when(cid == 0)
        def _():
            run(eh_r, True)

        @pl.when(cid == 1)
        def _():
            run(sh_r, False)

        plsc.subcore_barrier()

        @pl.when(cid == 0)
        def _():
            _row_copy(acc, out_se, sid, n)

            @pl.when(sid == 0)
            def _():
                pltpu.sync_copy(dacc, out_dg)

        @pl.when(cid == 1)
        def _():
            _row_copy(acc, out_ss, sid, n)

    return k1


def _make_k2(n, e):
    epc = e // NC
    ept = epc // NS
    ngrp = ept // C
    nblk = ngrp // NB
    rem = ngrp % NB
    na = _acc_rows(n)

    @functools.partial(
        pl.kernel,
        out_type=[
            jax.ShapeDtypeStruct((2, n, 128), F32),
        ],
        mesh=_sc_mesh(),
        scratch_types=[
            pltpu.VMEM_SHARED((na, 128), F32),
            pltpu.VMEM((NB, C), jnp.int32),
            pltpu.VMEM((NB, C), jnp.int32),
            pltpu.VMEM((NB, C, 128), F32),
            pltpu.VMEM((40, 128), F32),
            pltpu.SemaphoreType.DMA,
            pltpu.SemaphoreType.DMA,
            pltpu.SemaphoreType.DMA,
        ],
    )
    def k2(src_r, dst_r, tab_r, outp,
           acc, idxs, idxd, rows, zb, isem, gsem, ssem):
        cid = lax.axis_index("c")
        sid = lax.axis_index("s")
        _zero_acc(zb, acc, sid, n)
        plsc.subcore_barrier()

        base = cid * epc + sid * ept

        def start_loads(g, b):
            off = pl.multiple_of(base + g * C, 8)
            pltpu.async_copy(src_r.at[pl.ds(off, C)], idxs.at[b], isem)
            pltpu.async_copy(dst_r.at[pl.ds(off, C)], idxd.at[b], isem)

        def wait_loads(b):
            pltpu.make_async_copy(src_r.at[pl.ds(0, C)], idxs.at[b],
                                  isem).wait()
            pltpu.make_async_copy(dst_r.at[pl.ds(0, C)], idxd.at[b],
                                  isem).wait()

        def fire_gather(b):
            pltpu.async_copy(tab_r.at[idxs.at[b]], rows.at[b], gsem)

        def drain_gather(b):
            pltpu.make_async_copy(tab_r.at[idxs.at[b]], rows.at[b],
                                  gsem).wait()

        def fire_scatter(b):
            pltpu.async_copy(rows.at[b], acc.at[idxd.at[b]], ssem, add=True)

        def drain_scatter(b):
            pltpu.make_async_copy(rows.at[b], acc.at[idxd.at[b]],
                                  ssem).wait()

        for g in range(NB - 2):
            start_loads(g, g)

        def turn(g, b):
            wait_loads(b)
            fire_gather(b)

            @pl.when(g >= 1)
            def _():
                drain_gather((b + 3) % NB)
                fire_scatter((b + 3) % NB)

            @pl.when(g >= 2)
            def _():
                drain_scatter((b + 2) % NB)

            @pl.when(g + 2 < ngrp)
            def _():
                start_loads(g + 2, (b + 2) % NB)

        def blk(j, c):
            for b in range(NB):
                turn(j * NB + b, b)
            return c

        lax.fori_loop(0, nblk, blk, 0)
        for r in range(rem):
            g = ngrp - rem + r
            b = g % NB
            wait_loads(b)
            fire_gather(b)
            drain_gather((b + 3) % NB)
            fire_scatter((b + 3) % NB)
            drain_scatter((b + 2) % NB)
        bl = (ngrp - 1) % NB
        drain_gather(bl)
        fire_scatter(bl)
        drain_scatter((bl + 3) % NB)
        drain_scatter(bl)
        plsc.subcore_barrier()
        _row_copy(acc, outp.at[cid], sid, n)

    return k2


def _dotT(a, b):
    return lax.dot_general(a, b, (((1,), (1,)), ((), ())),
                           preferred_element_type=F32)


def _k3_body(se_ref, ss_ref, dg_ref, h_ref, tw_ref, iw_ref, tb_ref, ib_ref,
             out_ref):
    deg = dg_ref[...]
    degc = jnp.maximum(deg, 1.0)
    h_o_r = h_ref[...] * se_ref[...] / degc
    num = _dotT(ss_ref[...], tw_ref[...]) + deg * tb_ref[...]
    h_o_s = num / degc
    out_ref[...] = (_dotT(h_o_s, iw_ref[:, :128])
                    + _dotT(h_o_r, iw_ref[:, 128:])
                    + ib_ref[...])


def _make_k3(n):
    bn = 2000
    grid = (n // bn,)
    return pl.pallas_call(
        _k3_body,
        grid=grid,
        in_specs=[
            pl.BlockSpec((bn, 128), lambda i: (i, 0)),
            pl.BlockSpec((bn, 128), lambda i: (i, 0)),
            pl.BlockSpec((bn, 1), lambda i: (i, 0)),
            pl.BlockSpec((bn, 128), lambda i: (i, 0)),
            pl.BlockSpec((128, 128), lambda i: (0, 0)),
            pl.BlockSpec((128, 256), lambda i: (0, 0)),
            pl.BlockSpec((1, 128), lambda i: (0, 0)),
            pl.BlockSpec((1, 128), lambda i: (0, 0)),
        ],
        out_specs=pl.BlockSpec((bn, 128), lambda i: (i, 0)),
        out_shape=jax.ShapeDtypeStruct((n, 128), F32),
    )


def _k4_body(eh_ref, sh_ref, rw_ref, tw_ref, rb_ref, tb_ref, out_ref,
             wc_ref, bc_ref):
    @pl.when(pl.program_id(0) == 0)
    def _():
        rw2 = rw_ref[:, 128:]
        wc_ref[:128, :] = jnp.transpose(rw_ref[:, :128])
        wc_ref[128:, :] = lax.dot_general(tw_ref[...], rw2,
                                          (((0,), (1,)), ((), ())),
                                          preferred_element_type=F32)
        bc_ref[...] = rb_ref[...] + _dotT(tb_ref[...], rw2)

    x = jnp.concatenate([eh_ref[...], sh_ref[...]], axis=1)
    out_ref[...] = (jnp.dot(x, wc_ref[...], preferred_element_type=F32)
                    + bc_ref[...])


def _make_k4(e):
    be = 4000
    grid = (e // be,)
    return pl.pallas_call(
        _k4_body,
        grid=grid,
        in_specs=[
            pl.BlockSpec((be, 128), lambda i: (i, 0)),
            pl.BlockSpec((be, 128), lambda i: (i, 0)),
            pl.BlockSpec((128, 256), lambda i: (0, 0)),
            pl.BlockSpec((128, 128), lambda i: (0, 0)),
            pl.BlockSpec((1, 128), lambda i: (0, 0)),
            pl.BlockSpec((1, 128), lambda i: (0, 0)),
        ],
        out_specs=pl.BlockSpec((be, 128), lambda i: (i, 0)),
        out_shape=jax.ShapeDtypeStruct((e, 128), F32),
        scratch_shapes=[
            pltpu.VMEM((256, 128), F32),
            pltpu.VMEM((1, 128), F32),
        ],
    )


def _k5_body(pp_ref, nm_ref, bv_ref, out_ref):
    out_ref[...] = (pp_ref[0] + pp_ref[1]) * nm_ref[...] + bv_ref[...]


def _make_k5(n):
    bn = 2000
    grid = (n // bn,)
    return pl.pallas_call(
        _k5_body,
        grid=grid,
        in_specs=[
            pl.BlockSpec((2, bn, 128), lambda i: (0, i, 0)),
            pl.BlockSpec((bn, 1), lambda i: (i, 0)),
            pl.BlockSpec((1, 128), lambda i: (0, 0)),
        ],
        out_specs=pl.BlockSpec((bn, 128), lambda i: (i, 0)),
        out_shape=jax.ShapeDtypeStruct((n, 128), F32),
    )


def kernel(h, norm, e_h, s_h, edge_index, text_W, text_b, inv_W, inv_b,
           rel_W, rel_b, bias_v):
    n = h.shape[0]
    e = e_h.shape[0]
    src = edge_index[0]
    dst = edge_index[1]
    tb = text_b.reshape(1, 128)
    ib = inv_b.reshape(1, 128)
    rb = rel_b.reshape(1, 128)
    bv = bias_v.reshape(1, 128)

    sum_e, sum_s, deg = _make_k1(n, e)(dst, e_h, s_h)
    h_s_r_o = _make_k3(n)(sum_e, sum_s, deg.reshape(n, 1), h, text_W,
                          inv_W, tb, ib)
    (presum,) = _make_k2(n, e)(src, dst, h_s_r_o)
    h_new = _make_k5(n)(presum, norm, bv)
    e_h_new = _make_k4(e)(e_h, s_h, rel_W, text_W, rb, tb)
    return h_new, e_h_new


# speedup vs baseline: 9.8845x; 1.0016x over previous
import functools

import jax
import jax.numpy as jnp
from jax import lax
from jax.experimental import pallas as pl
from jax.experimental.pallas import tpu as pltpu
from jax.experimental.pallas import tpu_sc as plsc

F32 = jnp.float32

NC = 2
NS = 16
C = 80


def _sc_mesh():
    return plsc.VectorSubcoreMesh(core_axis_name="c", subcore_axis_name="s")


def _row_copy(src, dst, sid, n):
    per = (-(-n // NS) + 7) // 8 * 8
    last = n - (NS - 1) * per
    assert last > 0 and last % 8 == 0

    @pl.when(sid < NS - 1)
    def _():
        st = pl.multiple_of(sid * per, 8)
        pltpu.sync_copy(src.at[pl.ds(st, per), :], dst.at[pl.ds(st, per), :])

    @pl.when(sid == NS - 1)
    def _():
        st = (NS - 1) * per
        pltpu.sync_copy(src.at[pl.ds(st, last), :], dst.at[pl.ds(st, last), :])


def _zero_acc(zb, acc, sid, n):
    zh = 40

    def zrow(i, c):
        for t in range(8):
            zb[i, pl.ds(16 * t, 16)] = jnp.zeros((16,), F32)
        return c

    lax.fori_loop(0, zh, zrow, 0)
    per = (-(-n // NS) + zh - 1) // zh * zh
    last = n - (NS - 1) * per
    assert last > 0 and last % zh == 0

    @pl.when(sid < NS - 1)
    def _():
        st = pl.multiple_of(sid * per, 8)
        for q in range(per // zh):
            pltpu.sync_copy(zb, acc.at[pl.ds(st + q * zh, zh), :])

    @pl.when(sid == NS - 1)
    def _():
        st = (NS - 1) * per
        for q in range(last // zh):
            pltpu.sync_copy(zb, acc.at[pl.ds(st + q * zh, zh), :])


NB = 4


def _acc_rows(n):
    return -(-n // (NS * 64)) * (NS * 64)


def _make_k1(n, e):
    ept = e // NS
    ngrp = ept // C
    nblk = ngrp // NB
    rem = ngrp % NB
    na = _acc_rows(n)

    @functools.partial(
        pl.kernel,
        out_type=[
            jax.ShapeDtypeStruct((n, 128), F32),
            jax.ShapeDtypeStruct((n, 128), F32),
            jax.ShapeDtypeStruct((n,), F32),
        ],
        mesh=_sc_mesh(),
        scratch_types=[
            pltpu.VMEM_SHARED((na, 128), F32),
            pltpu.VMEM_SHARED((n,), F32),
            pltpu.VMEM((NB, C), jnp.int32),
            pltpu.VMEM((NB, C, 128), F32),
            pltpu.VMEM((C,), F32),
            pltpu.VMEM((40, 128), F32),
            pltpu.VMEM((640,), F32),
            pltpu.SemaphoreType.DMA,
            pltpu.SemaphoreType.DMA,
            pltpu.SemaphoreType.DMA,
            pltpu.SemaphoreType.DMA,
        ],
    )
    def k1(dst_r, eh_r, sh_r, out_se, out_ss, out_dg,
           acc, dacc, idxb, rows, ones_v, zb, zv, isem, rsem, ssem, dsem):
        cid = lax.axis_index("c")
        sid = lax.axis_index("s")
        for t in range(C // 16):
            ones_v[pl.ds(16 * t, 16)] = jnp.full((16,), 1.0, F32)
        _zero_acc(zb, acc, sid, n)

        def zvrow(i, c):
            zv[pl.ds(16 * i, 16)] = jnp.zeros((16,), F32)
            return c

        lax.fori_loop(0, 40, zvrow, 0)
        dper = 640
        dlast = n - (NS - 1) * dper
        assert 0 < dlast <= dper and dlast % 8 == 0

        @pl.when(sid < NS - 1)
        def _():
            st = pl.multiple_of(sid * dper, 8)
            pltpu.sync_copy(zv, dacc.at[pl.ds(st, dper)])

        @pl.when(sid == NS - 1)
        def _():
            pltpu.sync_copy(zv.at[pl.ds(0, dlast)],
                            dacc.at[pl.ds((NS - 1) * dper, dlast)])

        plsc.subcore_barrier()

        base = sid * ept

        def start_loads(arr_r, g, b):
            off = pl.multiple_of(base + g * C, 8)
            pltpu.async_copy(dst_r.at[pl.ds(off, C)], idxb.at[b], isem)
            pltpu.async_copy(arr_r.at[pl.ds(off, C), :], rows.at[b], rsem)

        def wait_loads(arr_r, b):
            pltpu.make_async_copy(dst_r.at[pl.ds(0, C)], idxb.at[b],
                                  isem).wait()
            pltpu.make_async_copy(arr_r.at[pl.ds(0, C), :], rows.at[b],
                                  rsem).wait()

        def fire_scatters(b, do_deg):
            pltpu.async_copy(rows.at[b], acc.at[idxb.at[b]], ssem, add=True)
            if do_deg:
                pltpu.async_copy(ones_v, dacc.at[idxb.at[b]], dsem, add=True)

        def drain_scatters(b, do_deg):
            pltpu.make_async_copy(rows.at[b], acc.at[idxb.at[b]], ssem).wait()
            if do_deg:
                pltpu.make_async_copy(ones_v, dacc.at[idxb.at[b]],
                                      dsem).wait()

        def run(arr_r, do_deg):
            for g in range(NB - 2):
                start_loads(arr_r, g, g)

            def turn(g, b):
                wait_loads(arr_r, b)

                @pl.when(g >= 2)
                def _():
                    drain_scatters((b + 2) % NB, do_deg)

                @pl.when(g + 2 < ngrp)
                def _():
                    start_loads(arr_r, g + 2, (b + 2) % NB)

                fire_scatters(b, do_deg)

            def blk(j, c):
                for b in range(NB):
                    turn(j * NB + b, b)
                return c

            lax.fori_loop(0, nblk, blk, 0)
            for r in range(rem):
                g = ngrp - rem + r
                b = g % NB
                wait_loads(arr_r, b)
                drain_scatters((b + 2) % NB, do_deg)
                fire_scatters(b, do_deg)
            drain_scatters((ngrp - 2) % NB, do_deg)
            drain_scatters((ngrp - 1) % NB, do_deg)

        @pl.when(cid == 0)
        def _():
            run(eh_r, True)

        @pl.when(cid == 1)
        def _():
            run(sh_r, False)

        plsc.subcore_barrier()

        @pl.when(cid == 0)
        def _():
            _row_copy(acc, out_se, sid, n)

            @pl.when(sid == 0)
            def _():
                pltpu.sync_copy(dacc, out_dg)

        @pl.when(cid == 1)
        def _():
            _row_copy(acc, out_ss, sid, n)

    return k1


def _make_k2(n, e):
    epc = e // NC
    ept = epc // NS
    ngrp = ept // C
    nblk = ngrp // NB
    rem = ngrp % NB
    na = _acc_rows(n)

    @functools.partial(
        pl.kernel,
        out_type=[
            jax.ShapeDtypeStruct((2, n, 128), F32),
        ],
        mesh=_sc_mesh(),
        scratch_types=[
            pltpu.VMEM_SHARED((na, 128), F32),
            pltpu.VMEM((NB, C), jnp.int32),
            pltpu.VMEM((NB, C), jnp.int32),
            pltpu.VMEM((NB, C, 128), F32),
            pltpu.VMEM((40, 128), F32),
            pltpu.SemaphoreType.DMA,
            pltpu.SemaphoreType.DMA,
            pltpu.SemaphoreType.DMA,
        ],
    )
    def k2(src_r, dst_r, tab_r, outp,
           acc, idxs, idxd, rows, zb, isem, gsem, ssem):
        cid = lax.axis_index("c")
        sid = lax.axis_index("s")
        _zero_acc(zb, acc, sid, n)
        plsc.subcore_barrier()

        base = cid * epc + sid * ept

        def start_loads(g, b):
            off = pl.multiple_of(base + g * C, 8)
            pltpu.async_copy(src_r.at[pl.ds(off, C)], idxs.at[b], isem)
            pltpu.async_copy(dst_r.at[pl.ds(off, C)], idxd.at[b], isem)

        def wait_loads(b):
            pltpu.make_async_copy(src_r.at[pl.ds(0, C)], idxs.at[b],
                                  isem).wait()
            pltpu.make_async_copy(dst_r.at[pl.ds(0, C)], idxd.at[b],
                                  isem).wait()

        def fire_gather(b):
            pltpu.async_copy(tab_r.at[idxs.at[b]], rows.at[b], gsem)

        def drain_gather(b):
            pltpu.make_async_copy(tab_r.at[idxs.at[b]], rows.at[b],
                                  gsem).wait()

        def fire_scatter(b):
            pltpu.async_copy(rows.at[b], acc.at[idxd.at[b]], ssem, add=True)

        def drain_scatter(b):
            pltpu.make_async_copy(rows.at[b], acc.at[idxd.at[b]],
                                  ssem).wait()

        for g in range(NB - 2):
            start_loads(g, g)

        def turn(g, b):
            wait_loads(b)
            fire_gather(b)

            @pl.when(g >= 1)
            def _():
                drain_gather((b + 3) % NB)
                fire_scatter((b + 3) % NB)

            @pl.when(g >= 2)
            def _():
                drain_scatter((b + 2) % NB)

            @pl.when(g + 2 < ngrp)
            def _():
                start_loads(g + 2, (b + 2) % NB)

        def blk(j, c):
            for b in range(NB):
                turn(j * NB + b, b)
            return c

        lax.fori_loop(0, nblk, blk, 0)
        for r in range(rem):
            g = ngrp - rem + r
            b = g % NB
            wait_loads(b)
            fire_gather(b)
            drain_gather((b + 3) % NB)
            fire_scatter((b + 3) % NB)
            drain_scatter((b + 2) % NB)
        bl = (ngrp - 1) % NB
        drain_gather(bl)
        fire_scatter(bl)
        drain_scatter((bl + 3) % NB)
        drain_scatter(bl)
        plsc.subcore_barrier()
        _row_copy(acc, outp.at[cid], sid, n)

    return k2


def _dotT(a, b):
    return lax.dot_general(a, b, (((1,), (1,)), ((), ())),
                           preferred_element_type=F32)


def _k3_body(se_ref, ss_ref, dg_ref, h_ref, tw_ref, iw_ref, tb_ref, ib_ref,
             out_ref):
    deg = dg_ref[...]
    degc = jnp.maximum(deg, 1.0)
    h_o_r = h_ref[...] * se_ref[...] / degc
    num = _dotT(ss_ref[...], tw_ref[...]) + deg * tb_ref[...]
    h_o_s = num / degc
    out_ref[...] = (_dotT(h_o_s, iw_ref[:, :128])
                    + _dotT(h_o_r, iw_ref[:, 128:])
                    + ib_ref[...])


def _make_k3(n):
    bn = 2000
    grid = (n // bn,)
    return pl.pallas_call(
        _k3_body,
        grid=grid,
        in_specs=[
            pl.BlockSpec((bn, 128), lambda i: (i, 0)),
            pl.BlockSpec((bn, 128), lambda i: (i, 0)),
            pl.BlockSpec((bn, 1), lambda i: (i, 0)),
            pl.BlockSpec((bn, 128), lambda i: (i, 0)),
            pl.BlockSpec((128, 128), lambda i: (0, 0)),
            pl.BlockSpec((128, 256), lambda i: (0, 0)),
            pl.BlockSpec((1, 128), lambda i: (0, 0)),
            pl.BlockSpec((1, 128), lambda i: (0, 0)),
        ],
        out_specs=pl.BlockSpec((bn, 128), lambda i: (i, 0)),
        out_shape=jax.ShapeDtypeStruct((n, 128), F32),
    )


def _k4_body(eh_ref, sh_ref, rw_ref, tw_ref, rb_ref, tb_ref, out_ref,
             wc_ref, bc_ref):
    @pl.when(pl.program_id(0) == 0)
    def _():
        rw2 = rw_ref[:, 128:]
        wc_ref[:128, :] = jnp.transpose(rw_ref[:, :128])
        wc_ref[128:, :] = lax.dot_general(tw_ref[...], rw2,
                                          (((0,), (1,)), ((), ())),
                                          preferred_element_type=F32)
        bc_ref[...] = rb_ref[...] + _dotT(tb_ref[...], rw2)

    x = jnp.concatenate([eh_ref[...], sh_ref[...]], axis=1)
    out_ref[...] = (jnp.dot(x, wc_ref[...], preferred_element_type=F32)
                    + bc_ref[...])


def _make_k4(e):
    be = 4000
    grid = (e // be,)
    return pl.pallas_call(
        _k4_body,
        grid=grid,
        in_specs=[
            pl.BlockSpec((be, 128), lambda i: (i, 0)),
            pl.BlockSpec((be, 128), lambda i: (i, 0)),
            pl.BlockSpec((128, 256), lambda i: (0, 0)),
            pl.BlockSpec((128, 128), lambda i: (0, 0)),
            pl.BlockSpec((1, 128), lambda i: (0, 0)),
            pl.BlockSpec((1, 128), lambda i: (0, 0)),
        ],
        out_specs=pl.BlockSpec((be, 128), lambda i: (i, 0)),
        out_shape=jax.ShapeDtypeStruct((e, 128), F32),
        scratch_shapes=[
            pltpu.VMEM((256, 128), F32),
            pltpu.VMEM((1, 128), F32),
        ],
    )


def _k5_body(pp_ref, nm_ref, bv_ref, out_ref):
    out_ref[...] = (pp_ref[0] + pp_ref[1]) * nm_ref[...] + bv_ref[...]


def _make_k5(n):
    bn = 2000
    grid = (n // bn,)
    return pl.pallas_call(
        _k5_body,
        grid=grid,
        in_specs=[
            pl.BlockSpec((2, bn, 128), lambda i: (0, i, 0)),
            pl.BlockSpec((bn, 1), lambda i: (i, 0)),
            pl.BlockSpec((1, 128), lambda i: (0, 0)),
        ],
        out_specs=pl.BlockSpec((bn, 128), lambda i: (i, 0)),
        out_shape=jax.ShapeDtypeStruct((n, 128), F32),
    )


def kernel(h, norm, e_h, s_h, edge_index, text_W, text_b, inv_W, inv_b,
           rel_W, rel_b, bias_v):
    n = h.shape[0]
    e = e_h.shape[0]
    src = edge_index[0]
    dst = edge_index[1]
    tb = text_b.reshape(1, 128)
    ib = inv_b.reshape(1, 128)
    rb = rel_b.reshape(1, 128)
    bv = bias_v.reshape(1, 128)

    e_h_new = _make_k4(e)(e_h, s_h, rel_W, text_W, rb, tb)
    sum_e, sum_s, deg = _make_k1(n, e)(dst, e_h, s_h)
    h_s_r_o = _make_k3(n)(sum_e, sum_s, deg.reshape(n, 1), h, text_W,
                          inv_W, tb, ib)
    (presum,) = _make_k2(n, e)(src, dst, h_s_r_o)
    h_new = _make_k5(n)(presum, norm, bv)
    return h_new, e_h_new


# speedup vs baseline: 9.9547x; 1.0071x over previous
import functools

import jax
import jax.numpy as jnp
from jax import lax
from jax.experimental import pallas as pl
from jax.experimental.pallas import tpu as pltpu
from jax.experimental.pallas import tpu_sc as plsc

F32 = jnp.float32

NC = 2
NS = 16
C = 80


def _sc_mesh():
    return plsc.VectorSubcoreMesh(core_axis_name="c", subcore_axis_name="s")


def _row_copy(src, dst, sid, n):
    per = (-(-n // NS) + 7) // 8 * 8
    last = n - (NS - 1) * per
    assert last > 0 and last % 8 == 0

    @pl.when(sid < NS - 1)
    def _():
        st = pl.multiple_of(sid * per, 8)
        pltpu.sync_copy(src.at[pl.ds(st, per), :], dst.at[pl.ds(st, per), :])

    @pl.when(sid == NS - 1)
    def _():
        st = (NS - 1) * per
        pltpu.sync_copy(src.at[pl.ds(st, last), :], dst.at[pl.ds(st, last), :])


def _zero_acc(zb, acc, sid, n):
    zh = 40

    def zrow(i, c):
        for t in range(8):
            zb[i, pl.ds(16 * t, 16)] = jnp.zeros((16,), F32)
        return c

    lax.fori_loop(0, zh, zrow, 0)
    per = (-(-n // NS) + zh - 1) // zh * zh
    last = n - (NS - 1) * per
    assert last > 0 and last % zh == 0

    @pl.when(sid < NS - 1)
    def _():
        st = pl.multiple_of(sid * per, 8)
        for q in range(per // zh):
            pltpu.sync_copy(zb, acc.at[pl.ds(st + q * zh, zh), :])

    @pl.when(sid == NS - 1)
    def _():
        st = (NS - 1) * per
        for q in range(last // zh):
            pltpu.sync_copy(zb, acc.at[pl.ds(st + q * zh, zh), :])


NB = 4


def _acc_rows(n):
    return -(-n // (NS * 64)) * (NS * 64)


def _make_k1(n, e):
    ept = e // NS
    ngrp = ept // C
    nblk = ngrp // NB
    rem = ngrp % NB
    na = _acc_rows(n)

    @functools.partial(
        pl.kernel,
        out_type=[
            jax.ShapeDtypeStruct((n, 128), F32),
            jax.ShapeDtypeStruct((n, 128), F32),
            jax.ShapeDtypeStruct((n,), F32),
        ],
        mesh=_sc_mesh(),
        scratch_types=[
            pltpu.VMEM_SHARED((na, 128), F32),
            pltpu.VMEM_SHARED((n,), F32),
            pltpu.VMEM((NB, C), jnp.int32),
            pltpu.VMEM((NB, C, 128), F32),
            pltpu.VMEM((C,), F32),
            pltpu.VMEM((40, 128), F32),
            pltpu.VMEM((640,), F32),
            pltpu.SemaphoreType.DMA,
            pltpu.SemaphoreType.DMA,
            pltpu.SemaphoreType.DMA,
            pltpu.SemaphoreType.DMA,
        ],
    )
    def k1(dst_r, eh_r, sh_r, out_se, out_ss, out_dg,
           acc, dacc, idxb, rows, ones_v, zb, zv, isem, rsem, ssem, dsem):
        cid = lax.axis_index("c")
        sid = lax.axis_index("s")
        for t in range(C // 16):
            ones_v[pl.ds(16 * t, 16)] = jnp.full((16,), 1.0, F32)
        _zero_acc(zb, acc, sid, n)

        def zvrow(i, c):
            zv[pl.ds(16 * i, 16)] = jnp.zeros((16,), F32)
            return c

        lax.fori_loop(0, 40, zvrow, 0)
        dper = 640
        dlast = n - (NS - 1) * dper
        assert 0 < dlast <= dper and dlast % 8 == 0

        @pl.when(sid < NS - 1)
        def _():
            st = pl.multiple_of(sid * dper, 8)
            pltpu.sync_copy(zv, dacc.at[pl.ds(st, dper)])

        @pl.when(sid == NS - 1)
        def _():
            pltpu.sync_copy(zv.at[pl.ds(0, dlast)],
                            dacc.at[pl.ds((NS - 1) * dper, dlast)])

        plsc.subcore_barrier()

        base = sid * ept

        def start_loads(arr_r, g, b):
            off = pl.multiple_of(base + g * C, 8)
            pltpu.async_copy(dst_r.at[pl.ds(off, C)], idxb.at[b], isem)
            pltpu.async_copy(arr_r.at[pl.ds(off, C), :], rows.at[b], rsem)

        def wait_loads(arr_r, b):
            pltpu.make_async_copy(dst_r.at[pl.ds(0, C)], idxb.at[b],
                                  isem).wait()
            pltpu.make_async_copy(arr_r.at[pl.ds(0, C), :], rows.at[b],
                                  rsem).wait()

        def fire_scatters(b, do_deg):
            pltpu.async_copy(rows.at[b], acc.at[idxb.at[b]], ssem, add=True)
            if do_deg:
                pltpu.async_copy(ones_v, dacc.at[idxb.at[b]], dsem, add=True)

        def drain_scatters(b, do_deg):
            pltpu.make_async_copy(rows.at[b], acc.at[idxb.at[b]], ssem).wait()
            if do_deg:
                pltpu.make_async_copy(ones_v, dacc.at[idxb.at[b]],
                                      dsem).wait()

        def run(arr_r, do_deg):
            for g in range(NB - 2):
                start_loads(arr_r, g, g)

            def turn(g, b):
                wait_loads(arr_r, b)

                @pl.when(g >= 2)
                def _():
                    drain_scatters((b + 2) % NB, do_deg)

                @pl.when(g + 2 < ngrp)
                def _():
                    start_loads(arr_r, g + 2, (b + 2) % NB)

                fire_scatters(b, do_deg)

            def blk(j, c):
                for b in range(NB):
                    turn(j * NB + b, b)
                return c

            lax.fori_loop(0, nblk, blk, 0)
            for r in range(rem):
                g = ngrp - rem + r
                b = g % NB
                wait_loads(arr_r, b)
                drain_scatters((b + 2) % NB, do_deg)
                fire_scatters(b, do_deg)
            drain_scatters((ngrp - 2) % NB, do_deg)
            drain_scatters((ngrp - 1) % NB, do_deg)

        @pl.when(cid == 0)
        def _():
            run(eh_r, True)

        @pl.when(cid == 1)
        def _():
            run(sh_r, False)

        plsc.subcore_barrier()

        @pl.when(cid == 0)
        def _():
            _row_copy(acc, out_se, sid, n)

            @pl.when(sid == 0)
            def _():
                pltpu.sync_copy(dacc, out_dg)

        @pl.when(cid == 1)
        def _():
            _row_copy(acc, out_ss, sid, n)

    return k1


def _make_k2(n, e):
    epc = e // NC
    ept = epc // NS
    ngrp = ept // C
    nblk = ngrp // NB
    rem = ngrp % NB
    na = _acc_rows(n)

    @functools.partial(
        pl.kernel,
        out_type=[
            jax.ShapeDtypeStruct((2, n, 128), F32),
        ],
        mesh=_sc_mesh(),
        scratch_types=[
            pltpu.VMEM_SHARED((na, 128), F32),
            pltpu.VMEM((NB, C), jnp.int32),
            pltpu.VMEM((NB, C), jnp.int32),
            pltpu.VMEM((NB, C, 128), F32),
            pltpu.VMEM((40, 128), F32),
            pltpu.SemaphoreType.DMA,
            pltpu.SemaphoreType.DMA,
            pltpu.SemaphoreType.DMA,
        ],
    )
    def k2(src_r, dst_r, tab_r, outp,
           acc, idxs, idxd, rows, zb, isem, gsem, ssem):
        cid = lax.axis_index("c")
        sid = lax.axis_index("s")
        _zero_acc(zb, acc, sid, n)
        plsc.subcore_barrier()

        base = cid * epc + sid * ept

        def start_loads(g, b):
            off = pl.multiple_of(base + g * C, 8)
            pltpu.async_copy(src_r.at[pl.ds(off, C)], idxs.at[b], isem)
            pltpu.async_copy(dst_r.at[pl.ds(off, C)], idxd.at[b], isem)

        def wait_loads(b):
            pltpu.make_async_copy(src_r.at[pl.ds(0, C)], idxs.at[b],
                                  isem).wait()
            pltpu.make_async_copy(dst_r.at[pl.ds(0, C)], idxd.at[b],
                                  isem).wait()

        def fire_gather(b):
            pltpu.async_copy(tab_r.at[idxs.at[b]], rows.at[b], gsem)

        def drain_gather(b):
            pltpu.make_async_copy(tab_r.at[idxs.at[b]], rows.at[b],
                                  gsem).wait()

        def fire_scatter(b):
            pltpu.async_copy(rows.at[b], acc.at[idxd.at[b]], ssem, add=True)

        def drain_scatter(b):
            pltpu.make_async_copy(rows.at[b], acc.at[idxd.at[b]],
                                  ssem).wait()

        for g in range(NB - 2):
            start_loads(g, g)

        def turn(g, b):
            wait_loads(b)
            fire_gather(b)

            @pl.when(g >= 1)
            def _():
                drain_gather((b + 3) % NB)
                fire_scatter((b + 3) % NB)

            @pl.when(g >= 2)
            def _():
                drain_scatter((b + 2) % NB)

            @pl.when(g + 2 < ngrp)
            def _():
                start_loads(g + 2, (b + 2) % NB)

        def blk(j, c):
            for b in range(NB):
                turn(j * NB + b, b)
            return c

        lax.fori_loop(0, nblk, blk, 0)
        for r in range(rem):
            g = ngrp - rem + r
            b = g % NB
            wait_loads(b)
            fire_gather(b)
            drain_gather((b + 3) % NB)
            fire_scatter((b + 3) % NB)
            drain_scatter((b + 2) % NB)
        bl = (ngrp - 1) % NB
        drain_gather(bl)
        fire_scatter(bl)
        drain_scatter((bl + 3) % NB)
        drain_scatter(bl)
        plsc.subcore_barrier()
        _row_copy(acc, outp.at[cid], sid, n)

    return k2


def _dotT(a, b):
    return lax.dot_general(a, b, (((1,), (1,)), ((), ())),
                           preferred_element_type=F32)


def _k3_body(se_ref, ss_ref, dg_ref, h_ref, tw_ref, iw_ref, tb_ref, ib_ref,
             out_ref):
    deg = dg_ref[...]
    degc = jnp.maximum(deg, 1.0)
    h_o_r = h_ref[...] * se_ref[...] / degc
    num = _dotT(ss_ref[...], tw_ref[...]) + deg * tb_ref[...]
    h_o_s = num / degc
    out_ref[...] = (_dotT(h_o_s, iw_ref[:, :128])
                    + _dotT(h_o_r, iw_ref[:, 128:])
                    + ib_ref[...])


def _make_k3(n):
    bn = 2000
    grid = (n // bn,)
    return pl.pallas_call(
        _k3_body,
        grid=grid,
        in_specs=[
            pl.BlockSpec((bn, 128), lambda i: (i, 0)),
            pl.BlockSpec((bn, 128), lambda i: (i, 0)),
            pl.BlockSpec((bn, 1), lambda i: (i, 0)),
            pl.BlockSpec((bn, 128), lambda i: (i, 0)),
            pl.BlockSpec((128, 128), lambda i: (0, 0)),
            pl.BlockSpec((128, 256), lambda i: (0, 0)),
            pl.BlockSpec((1, 128), lambda i: (0, 0)),
            pl.BlockSpec((1, 128), lambda i: (0, 0)),
        ],
        out_specs=pl.BlockSpec((bn, 128), lambda i: (i, 0)),
        out_shape=jax.ShapeDtypeStruct((n, 128), F32),
    )


def _k4_body(eh_ref, sh_ref, rw_ref, tw_ref, rb_ref, tb_ref, out_ref,
             wc_ref, bc_ref):
    @pl.when(pl.program_id(0) == 0)
    def _():
        rw2 = rw_ref[:, 128:]
        wc_ref[:128, :] = jnp.transpose(rw_ref[:, :128])
        wc_ref[128:, :] = lax.dot_general(tw_ref[...], rw2,
                                          (((0,), (1,)), ((), ())),
                                          preferred_element_type=F32)
        bc_ref[...] = rb_ref[...] + _dotT(tb_ref[...], rw2)

    x = jnp.concatenate([eh_ref[...], sh_ref[...]], axis=1)
    out_ref[...] = (jnp.dot(x, wc_ref[...], preferred_element_type=F32)
                    + bc_ref[...])


def _make_k4(e):
    be = 8000
    grid = (e // be,)
    return pl.pallas_call(
        _k4_body,
        grid=grid,
        in_specs=[
            pl.BlockSpec((be, 128), lambda i: (i, 0)),
            pl.BlockSpec((be, 128), lambda i: (i, 0)),
            pl.BlockSpec((128, 256), lambda i: (0, 0)),
            pl.BlockSpec((128, 128), lambda i: (0, 0)),
            pl.BlockSpec((1, 128), lambda i: (0, 0)),
            pl.BlockSpec((1, 128), lambda i: (0, 0)),
        ],
        out_specs=pl.BlockSpec((be, 128), lambda i: (i, 0)),
        out_shape=jax.ShapeDtypeStruct((e, 128), F32),
        scratch_shapes=[
            pltpu.VMEM((256, 128), F32),
            pltpu.VMEM((1, 128), F32),
        ],
    )


def _k5_body(pp_ref, nm_ref, bv_ref, out_ref):
    out_ref[...] = (pp_ref[0] + pp_ref[1]) * nm_ref[...] + bv_ref[...]


def _make_k5(n):
    bn = 2000
    grid = (n // bn,)
    return pl.pallas_call(
        _k5_body,
        grid=grid,
        in_specs=[
            pl.BlockSpec((2, bn, 128), lambda i: (0, i, 0)),
            pl.BlockSpec((bn, 1), lambda i: (i, 0)),
            pl.BlockSpec((1, 128), lambda i: (0, 0)),
        ],
        out_specs=pl.BlockSpec((bn, 128), lambda i: (i, 0)),
        out_shape=jax.ShapeDtypeStruct((n, 128), F32),
    )


def kernel(h, norm, e_h, s_h, edge_index, text_W, text_b, inv_W, inv_b,
           rel_W, rel_b, bias_v):
    n = h.shape[0]
    e = e_h.shape[0]
    src = edge_index[0]
    dst = edge_index[1]
    tb = text_b.reshape(1, 128)
    ib = inv_b.reshape(1, 128)
    rb = rel_b.reshape(1, 128)
    bv = bias_v.reshape(1, 128)

    e_h_new = _make_k4(e)(e_h, s_h, rel_W, text_W, rb, tb)
    sum_e, sum_s, deg = _make_k1(n, e)(dst, e_h, s_h)
    h_s_r_o = _make_k3(n)(sum_e, sum_s, deg.reshape(n, 1), h, text_W,
                          inv_W, tb, ib)
    (presum,) = _make_k2(n, e)(src, dst, h_s_r_o)
    h_new = _make_k5(n)(presum, norm, bv)
    return h_new, e_h_new


# speedup vs baseline: 9.9961x; 1.0042x over previous
import functools

import jax
import jax.numpy as jnp
from jax import lax
from jax.experimental import pallas as pl
from jax.experimental.pallas import tpu as pltpu
from jax.experimental.pallas import tpu_sc as plsc

F32 = jnp.float32

NC = 2
NS = 16
C = 80


def _sc_mesh():
    return plsc.VectorSubcoreMesh(core_axis_name="c", subcore_axis_name="s")


def _row_copy(src, dst, sid, n):
    per = (-(-n // NS) + 7) // 8 * 8
    last = n - (NS - 1) * per
    assert last > 0 and last % 8 == 0

    @pl.when(sid < NS - 1)
    def _():
        st = pl.multiple_of(sid * per, 8)
        pltpu.sync_copy(src.at[pl.ds(st, per), :], dst.at[pl.ds(st, per), :])

    @pl.when(sid == NS - 1)
    def _():
        st = (NS - 1) * per
        pltpu.sync_copy(src.at[pl.ds(st, last), :], dst.at[pl.ds(st, last), :])


def _zero_acc(zb, acc, sid, n):
    zh = 40

    def zrow(i, c):
        for t in range(8):
            zb[i, pl.ds(16 * t, 16)] = jnp.zeros((16,), F32)
        return c

    lax.fori_loop(0, zh, zrow, 0)
    per = (-(-n // NS) + zh - 1) // zh * zh
    last = n - (NS - 1) * per
    assert last > 0 and last % zh == 0

    @pl.when(sid < NS - 1)
    def _():
        st = pl.multiple_of(sid * per, 8)
        for q in range(per // zh):
            pltpu.sync_copy(zb, acc.at[pl.ds(st + q * zh, zh), :])

    @pl.when(sid == NS - 1)
    def _():
        st = (NS - 1) * per
        for q in range(last // zh):
            pltpu.sync_copy(zb, acc.at[pl.ds(st + q * zh, zh), :])


NB = 4


def _acc_rows(n):
    return -(-n // (NS * 64)) * (NS * 64)


def _make_k1(n, e):
    ept = e // NS
    ngrp = ept // C
    nblk = ngrp // NB
    rem = ngrp % NB
    na = _acc_rows(n)

    @functools.partial(
        pl.kernel,
        out_type=[
            jax.ShapeDtypeStruct((n, 128), F32),
            jax.ShapeDtypeStruct((n, 128), F32),
            jax.ShapeDtypeStruct((n,), F32),
        ],
        mesh=_sc_mesh(),
        scratch_types=[
            pltpu.VMEM_SHARED((na, 128), F32),
            pltpu.VMEM_SHARED((n,), F32),
            pltpu.VMEM((NB, C), jnp.int32),
            pltpu.VMEM((NB, C, 128), F32),
            pltpu.VMEM((C,), F32),
            pltpu.VMEM((40, 128), F32),
            pltpu.VMEM((640,), F32),
            pltpu.SemaphoreType.DMA,
            pltpu.SemaphoreType.DMA,
            pltpu.SemaphoreType.DMA,
            pltpu.SemaphoreType.DMA,
        ],
    )
    def k1(dst_r, eh_r, sh_r, out_se, out_ss, out_dg,
           acc, dacc, idxb, rows, ones_v, zb, zv, isem, rsem, ssem, dsem):
        cid = lax.axis_index("c")
        sid = lax.axis_index("s")
        for t in range(C // 16):
            ones_v[pl.ds(16 * t, 16)] = jnp.full((16,), 1.0, F32)
        _zero_acc(zb, acc, sid, n)

        def zvrow(i, c):
            zv[pl.ds(16 * i, 16)] = jnp.zeros((16,), F32)
            return c

        lax.fori_loop(0, 40, zvrow, 0)
        dper = 640
        dlast = n - (NS - 1) * dper
        assert 0 < dlast <= dper and dlast % 8 == 0

        @pl.when(sid < NS - 1)
        def _():
            st = pl.multiple_of(sid * dper, 8)
            pltpu.sync_copy(zv, dacc.at[pl.ds(st, dper)])

        @pl.when(sid == NS - 1)
        def _():
            pltpu.sync_copy(zv.at[pl.ds(0, dlast)],
                            dacc.at[pl.ds((NS - 1) * dper, dlast)])

        plsc.subcore_barrier()

        base = sid * ept

        def start_loads(arr_r, g, b):
            off = pl.multiple_of(base + g * C, 8)
            pltpu.async_copy(dst_r.at[pl.ds(off, C)], idxb.at[b], isem)
            pltpu.async_copy(arr_r.at[pl.ds(off, C), :], rows.at[b], rsem)

        def wait_loads(arr_r, b):
            pltpu.make_async_copy(dst_r.at[pl.ds(0, C)], idxb.at[b],
                                  isem).wait()
            pltpu.make_async_copy(arr_r.at[pl.ds(0, C), :], rows.at[b],
                                  rsem).wait()

        def fire_scatters(b, do_deg):
            pltpu.async_copy(rows.at[b], acc.at[idxb.at[b]], ssem, add=True)
            if do_deg:
                pltpu.async_copy(ones_v, dacc.at[idxb.at[b]], dsem, add=True)

        def drain_scatters(b, do_deg):
            pltpu.make_async_copy(rows.at[b], acc.at[idxb.at[b]], ssem).wait()
            if do_deg:
                pltpu.make_async_copy(ones_v, dacc.at[idxb.at[b]],
                                      dsem).wait()

        def run(arr_r, do_deg):
            for g in range(NB - 2):
                start_loads(arr_r, g, g)

            def turn(g, b):
                wait_loads(arr_r, b)

                @pl.when(g >= 2)
                def _():
                    drain_scatters((b + 2) % NB, do_deg)

                @pl.when(g + 2 < ngrp)
                def _():
                    start_loads(arr_r, g + 2, (b + 2) % NB)

                fire_scatters(b, do_deg)

            def blk(j, c):
                for b in range(NB):
                    turn(j * NB + b, b)
                return c

            lax.fori_loop(0, nblk, blk, 0)
            for r in range(rem):
                g = ngrp - rem + r
                b = g % NB
                wait_loads(arr_r, b)
                drain_scatters((b + 2) % NB, do_deg)
                fire_scatters(b, do_deg)
            drain_scatters((ngrp - 2) % NB, do_deg)
            drain_scatters((ngrp - 1) % NB, do_deg)

        @pl.when(cid == 0)
        def _():
            run(eh_r, True)

        @pl.when(cid == 1)
        def _():
            run(sh_r, False)

        plsc.subcore_barrier()

        @pl.when(cid == 0)
        def _():
            _row_copy(acc, out_se, sid, n)

            @pl.when(sid == 0)
            def _():
                pltpu.sync_copy(dacc, out_dg)

        @pl.when(cid == 1)
        def _():
            _row_copy(acc, out_ss, sid, n)

    return k1


def _make_k2(n, e):
    epc = e // NC
    ept = epc // NS
    ngrp = ept // C
    nblk = ngrp // NB
    rem = ngrp % NB
    na = _acc_rows(n)

    @functools.partial(
        pl.kernel,
        out_type=[
            jax.ShapeDtypeStruct((2, n, 128), F32),
        ],
        mesh=_sc_mesh(),
        scratch_types=[
            pltpu.VMEM_SHARED((na, 128), F32),
            pltpu.VMEM((NB, C), jnp.int32),
            pltpu.VMEM((NB, C), jnp.int32),
            pltpu.VMEM((NB, C, 128), F32),
            pltpu.VMEM((40, 128), F32),
            pltpu.SemaphoreType.DMA,
            pltpu.SemaphoreType.DMA,
            pltpu.SemaphoreType.DMA,
        ],
    )
    def k2(src_r, dst_r, tab_r, outp,
           acc, idxs, idxd, rows, zb, isem, gsem, ssem):
        cid = lax.axis_index("c")
        sid = lax.axis_index("s")
        _zero_acc(zb, acc, sid, n)
        plsc.subcore_barrier()

        base = cid * epc + sid * ept

        def start_loads(g, b):
            off = pl.multiple_of(base + g * C, 8)
            pltpu.async_copy(src_r.at[pl.ds(off, C)], idxs.at[b], isem)
            pltpu.async_copy(dst_r.at[pl.ds(off, C)], idxd.at[b], isem)

        def wait_loads(b):
            pltpu.make_async_copy(src_r.at[pl.ds(0, C)], idxs.at[b],
                                  isem).wait()
            pltpu.make_async_copy(dst_r.at[pl.ds(0, C)], idxd.at[b],
                                  isem).wait()

        def fire_gather(b):
            pltpu.async_copy(tab_r.at[idxs.at[b]], rows.at[b], gsem)

        def drain_gather(b):
            pltpu.make_async_copy(tab_r.at[idxs.at[b]], rows.at[b],
                                  gsem).wait()

        def fire_scatter(b):
            pltpu.async_copy(rows.at[b], acc.at[idxd.at[b]], ssem, add=True)

        def drain_scatter(b):
            pltpu.make_async_copy(rows.at[b], acc.at[idxd.at[b]],
                                  ssem).wait()

        for g in range(NB - 2):
            start_loads(g, g)

        def turn(g, b):
            wait_loads(b)
            fire_gather(b)

            @pl.when(g >= 1)
            def _():
                drain_gather((b + 3) % NB)
                fire_scatter((b + 3) % NB)

            @pl.when(g >= 2)
            def _():
                drain_scatter((b + 2) % NB)

            @pl.when(g + 2 < ngrp)
            def _():
                start_loads(g + 2, (b + 2) % NB)

        def blk(j, c):
            for b in range(NB):
                turn(j * NB + b, b)
            return c

        lax.fori_loop(0, nblk, blk, 0)
        for r in range(rem):
            g = ngrp - rem + r
            b = g % NB
            wait_loads(b)
            fire_gather(b)
            drain_gather((b + 3) % NB)
            fire_scatter((b + 3) % NB)
            drain_scatter((b + 2) % NB)
        bl = (ngrp - 1) % NB
        drain_gather(bl)
        fire_scatter(bl)
        drain_scatter((bl + 3) % NB)
        drain_scatter(bl)
        plsc.subcore_barrier()
        _row_copy(acc, outp.at[cid], sid, n)

    return k2


def _dotT(a, b):
    return lax.dot_general(a, b, (((1,), (1,)), ((), ())),
                           preferred_element_type=F32)


def _k3_body(se_ref, ss_ref, dg_ref, h_ref, tw_ref, iw_ref, tb_ref, ib_ref,
             out_ref):
    deg = dg_ref[...]
    degc = jnp.maximum(deg, 1.0)
    h_o_r = h_ref[...] * se_ref[...] / degc
    num = _dotT(ss_ref[...], tw_ref[...]) + deg * tb_ref[...]
    h_o_s = num / degc
    out_ref[...] = (_dotT(h_o_s, iw_ref[:, :128])
                    + _dotT(h_o_r, iw_ref[:, 128:])
                    + ib_ref[...])


def _make_k3(n):
    bn = 5000
    grid = (n // bn,)
    return pl.pallas_call(
        _k3_body,
        grid=grid,
        in_specs=[
            pl.BlockSpec((bn, 128), lambda i: (i, 0)),
            pl.BlockSpec((bn, 128), lambda i: (i, 0)),
            pl.BlockSpec((bn, 1), lambda i: (i, 0)),
            pl.BlockSpec((bn, 128), lambda i: (i, 0)),
            pl.BlockSpec((128, 128), lambda i: (0, 0)),
            pl.BlockSpec((128, 256), lambda i: (0, 0)),
            pl.BlockSpec((1, 128), lambda i: (0, 0)),
            pl.BlockSpec((1, 128), lambda i: (0, 0)),
        ],
        out_specs=pl.BlockSpec((bn, 128), lambda i: (i, 0)),
        out_shape=jax.ShapeDtypeStruct((n, 128), F32),
    )


def _k4_body(eh_ref, sh_ref, rw_ref, tw_ref, rb_ref, tb_ref, out_ref,
             wc_ref, bc_ref):
    @pl.when(pl.program_id(0) == 0)
    def _():
        rw2 = rw_ref[:, 128:]
        wc_ref[:128, :] = jnp.transpose(rw_ref[:, :128])
        wc_ref[128:, :] = lax.dot_general(tw_ref[...], rw2,
                                          (((0,), (1,)), ((), ())),
                                          preferred_element_type=F32)
        bc_ref[...] = rb_ref[...] + _dotT(tb_ref[...], rw2)

    x = jnp.concatenate([eh_ref[...], sh_ref[...]], axis=1)
    out_ref[...] = (jnp.dot(x, wc_ref[...], preferred_element_type=F32)
                    + bc_ref[...])


def _make_k4(e):
    be = 8000
    grid = (e // be,)
    return pl.pallas_call(
        _k4_body,
        grid=grid,
        in_specs=[
            pl.BlockSpec((be, 128), lambda i: (i, 0)),
            pl.BlockSpec((be, 128), lambda i: (i, 0)),
            pl.BlockSpec((128, 256), lambda i: (0, 0)),
            pl.BlockSpec((128, 128), lambda i: (0, 0)),
            pl.BlockSpec((1, 128), lambda i: (0, 0)),
            pl.BlockSpec((1, 128), lambda i: (0, 0)),
        ],
        out_specs=pl.BlockSpec((be, 128), lambda i: (i, 0)),
        out_shape=jax.ShapeDtypeStruct((e, 128), F32),
        scratch_shapes=[
            pltpu.VMEM((256, 128), F32),
            pltpu.VMEM((1, 128), F32),
        ],
    )


def _k5_body(pp_ref, nm_ref, bv_ref, out_ref):
    out_ref[...] = (pp_ref[0] + pp_ref[1]) * nm_ref[...] + bv_ref[...]


def _make_k5(n):
    bn = 5000
    grid = (n // bn,)
    return pl.pallas_call(
        _k5_body,
        grid=grid,
        in_specs=[
            pl.BlockSpec((2, bn, 128), lambda i: (0, i, 0)),
            pl.BlockSpec((bn, 1), lambda i: (i, 0)),
            pl.BlockSpec((1, 128), lambda i: (0, 0)),
        ],
        out_specs=pl.BlockSpec((bn, 128), lambda i: (i, 0)),
        out_shape=jax.ShapeDtypeStruct((n, 128), F32),
    )


def kernel(h, norm, e_h, s_h, edge_index, text_W, text_b, inv_W, inv_b,
           rel_W, rel_b, bias_v):
    n = h.shape[0]
    e = e_h.shape[0]
    src = edge_index[0]
    dst = edge_index[1]
    tb = text_b.reshape(1, 128)
    ib = inv_b.reshape(1, 128)
    rb = rel_b.reshape(1, 128)
    bv = bias_v.reshape(1, 128)

    e_h_new = _make_k4(e)(e_h, s_h, rel_W, text_W, rb, tb)
    sum_e, sum_s, deg = _make_k1(n, e)(dst, e_h, s_h)
    h_s_r_o = _make_k3(n)(sum_e, sum_s, deg.reshape(n, 1), h, text_W,
                          inv_W, tb, ib)
    (presum,) = _make_k2(n, e)(src, dst, h_s_r_o)
    h_new = _make_k5(n)(presum, norm, bv)
    return h_new, e_h_new


# speedup vs baseline: 10.0349x; 1.0039x over previous
import functools

import jax
import jax.numpy as jnp
from jax import lax
from jax.experimental import pallas as pl
from jax.experimental.pallas import tpu as pltpu
from jax.experimental.pallas import tpu_sc as plsc

F32 = jnp.float32

NC = 2
NS = 16
C = 80


def _sc_mesh():
    return plsc.VectorSubcoreMesh(core_axis_name="c", subcore_axis_name="s")


def _row_copy(src, dst, sid, n):
    per = (-(-n // NS) + 7) // 8 * 8
    last = n - (NS - 1) * per
    assert last > 0 and last % 8 == 0

    @pl.when(sid < NS - 1)
    def _():
        st = pl.multiple_of(sid * per, 8)
        pltpu.sync_copy(src.at[pl.ds(st, per), :], dst.at[pl.ds(st, per), :])

    @pl.when(sid == NS - 1)
    def _():
        st = (NS - 1) * per
        pltpu.sync_copy(src.at[pl.ds(st, last), :], dst.at[pl.ds(st, last), :])


def _zero_acc(zb, acc, sid, n):
    zh = 40

    def zrow(i, c):
        for t in range(8):
            zb[i, pl.ds(16 * t, 16)] = jnp.zeros((16,), F32)
        return c

    lax.fori_loop(0, zh, zrow, 0)
    per = (-(-n // NS) + zh - 1) // zh * zh
    last = n - (NS - 1) * per
    assert last > 0 and last % zh == 0

    @pl.when(sid < NS - 1)
    def _():
        st = pl.multiple_of(sid * per, 8)
        for q in range(per // zh):
            pltpu.sync_copy(zb, acc.at[pl.ds(st + q * zh, zh), :])

    @pl.when(sid == NS - 1)
    def _():
        st = (NS - 1) * per
        for q in range(last // zh):
            pltpu.sync_copy(zb, acc.at[pl.ds(st + q * zh, zh), :])


NB = 4


def _acc_rows(n):
    return -(-n // (NS * 64)) * (NS * 64)


def _make_k1(n, e):
    ept = e // NS
    ngrp = ept // C
    nblk = ngrp // NB
    rem = ngrp % NB
    na = _acc_rows(n)

    @functools.partial(
        pl.kernel,
        out_type=[
            jax.ShapeDtypeStruct((n, 128), F32),
            jax.ShapeDtypeStruct((n, 128), F32),
            jax.ShapeDtypeStruct((n,), F32),
        ],
        mesh=_sc_mesh(),
        scratch_types=[
            pltpu.VMEM_SHARED((na, 128), F32),
            pltpu.VMEM_SHARED((n,), F32),
            pltpu.VMEM((NB, C), jnp.int32),
            pltpu.VMEM((NB, C, 128), F32),
            pltpu.VMEM((C,), F32),
            pltpu.VMEM((40, 128), F32),
            pltpu.VMEM((640,), F32),
            pltpu.SemaphoreType.DMA,
            pltpu.SemaphoreType.DMA,
            pltpu.SemaphoreType.DMA,
            pltpu.SemaphoreType.DMA,
        ],
    )
    def k1(dst_r, eh_r, sh_r, out_se, out_ss, out_dg,
           acc, dacc, idxb, rows, ones_v, zb, zv, isem, rsem, ssem, dsem):
        cid = lax.axis_index("c")
        sid = lax.axis_index("s")
        for t in range(C // 16):
            ones_v[pl.ds(16 * t, 16)] = jnp.full((16,), 1.0, F32)
        _zero_acc(zb, acc, sid, n)

        def zvrow(i, c):
            zv[pl.ds(16 * i, 16)] = jnp.zeros((16,), F32)
            return c

        lax.fori_loop(0, 40, zvrow, 0)
        dper = 640
        dlast = n - (NS - 1) * dper
        assert 0 < dlast <= dper and dlast % 8 == 0

        @pl.when(sid < NS - 1)
        def _():
            st = pl.multiple_of(sid * dper, 8)
            pltpu.sync_copy(zv, dacc.at[pl.ds(st, dper)])

        @pl.when(sid == NS - 1)
        def _():
            pltpu.sync_copy(zv.at[pl.ds(0, dlast)],
                            dacc.at[pl.ds((NS - 1) * dper, dlast)])

        plsc.subcore_barrier()

        base = sid * ept

        def start_loads(arr_r, g, b):
            off = pl.multiple_of(base + g * C, 8)
            pltpu.async_copy(dst_r.at[pl.ds(off, C)], idxb.at[b], isem)
            pltpu.async_copy(arr_r.at[pl.ds(off, C), :], rows.at[b], rsem)

        def wait_loads(arr_r, b):
            pltpu.make_async_copy(dst_r.at[pl.ds(0, C)], idxb.at[b],
                                  isem).wait()
            pltpu.make_async_copy(arr_r.at[pl.ds(0, C), :], rows.at[b],
                                  rsem).wait()

        def fire_scatters(b, do_deg):
            pltpu.async_copy(rows.at[b], acc.at[idxb.at[b]], ssem, add=True)
            if do_deg:
                pltpu.async_copy(ones_v, dacc.at[idxb.at[b]], dsem, add=True)

        def drain_scatters(b, do_deg):
            pltpu.make_async_copy(rows.at[b], acc.at[idxb.at[b]], ssem).wait()
            if do_deg:
                pltpu.make_async_copy(ones_v, dacc.at[idxb.at[b]],
                                      dsem).wait()

        def run(arr_r, do_deg):
            for g in range(NB - 2):
                start_loads(arr_r, g, g)

            def turn(g, b):
                wait_loads(arr_r, b)

                @pl.when(g >= 2)
                def _():
                    drain_scatters((b + 2) % NB, do_deg)

                @pl.when(g + 2 < ngrp)
                def _():
                    start_loads(arr_r, g + 2, (b + 2) % NB)

                fire_scatters(b, do_deg)

            def blk(j, c):
                for b in range(NB):
                    turn(j * NB + b, b)
                return c

            lax.fori_loop(0, nblk, blk, 0)
            for r in range(rem):
                g = ngrp - rem + r
                b = g % NB
                wait_loads(arr_r, b)
                drain_scatters((b + 2) % NB, do_deg)
                fire_scatters(b, do_deg)
            drain_scatters((ngrp - 2) % NB, do_deg)
            drain_scatters((ngrp - 1) % NB, do_deg)

        @pl.when(cid == 0)
        def _():
            run(eh_r, True)

        @pl.when(cid == 1)
        def _():
            run(sh_r, False)

        plsc.subcore_barrier()

        @pl.when(cid == 0)
        def _():
            _row_copy(acc, out_se, sid, n)

            @pl.when(sid == 0)
            def _():
                pltpu.sync_copy(dacc, out_dg)

        @pl.when(cid == 1)
        def _():
            _row_copy(acc, out_ss, sid, n)

    return k1


def _make_k2(n, e):
    epc = e // NC
    ept = epc // NS
    ngrp = ept // C
    nblk = ngrp // NB
    rem = ngrp % NB
    na = _acc_rows(n)

    @functools.partial(
        pl.kernel,
        out_type=[
            jax.ShapeDtypeStruct((2, n, 128), F32),
        ],
        mesh=_sc_mesh(),
        scratch_types=[
            pltpu.VMEM_SHARED((na, 128), F32),
            pltpu.VMEM((NB, C), jnp.int32),
            pltpu.VMEM((NB, C), jnp.int32),
            pltpu.VMEM((NB, C, 128), F32),
            pltpu.VMEM((40, 128), F32),
            pltpu.SemaphoreType.DMA,
            pltpu.SemaphoreType.DMA,
            pltpu.SemaphoreType.DMA,
        ],
    )
    def k2(src_r, dst_r, tab_r, outp,
           acc, idxs, idxd, rows, zb, isem, gsem, ssem):
        cid = lax.axis_index("c")
        sid = lax.axis_index("s")
        _zero_acc(zb, acc, sid, n)
        plsc.subcore_barrier()

        base = cid * epc + sid * ept

        def start_loads(g, b):
            off = pl.multiple_of(base + g * C, 8)
            pltpu.async_copy(src_r.at[pl.ds(off, C)], idxs.at[b], isem)
            pltpu.async_copy(dst_r.at[pl.ds(off, C)], idxd.at[b], isem)

        def wait_loads(b):
            pltpu.make_async_copy(src_r.at[pl.ds(0, C)], idxs.at[b],
                                  isem).wait()
            pltpu.make_async_copy(dst_r.at[pl.ds(0, C)], idxd.at[b],
                                  isem).wait()

        def fire_gather(b):
            pltpu.async_copy(tab_r.at[idxs.at[b]], rows.at[b], gsem)

        def drain_gather(b):
            pltpu.make_async_copy(tab_r.at[idxs.at[b]], rows.at[b],
                                  gsem).wait()

        def fire_scatter(b):
            pltpu.async_copy(rows.at[b], acc.at[idxd.at[b]], ssem, add=True)

        def drain_scatter(b):
            pltpu.make_async_copy(rows.at[b], acc.at[idxd.at[b]],
                                  ssem).wait()

        for g in range(NB - 2):
            start_loads(g, g)

        def turn(g, b):
            wait_loads(b)
            fire_gather(b)

            @pl.when(g >= 1)
            def _():
                drain_gather((b + 3) % NB)
                fire_scatter((b + 3) % NB)

            @pl.when(g >= 2)
            def _():
                drain_scatter((b + 2) % NB)

            @pl.when(g + 2 < ngrp)
            def _():
                start_loads(g + 2, (b + 2) % NB)

        def blk(j, c):
            for b in range(NB):
                turn(j * NB + b, b)
            return c

        lax.fori_loop(0, nblk, blk, 0)
        for r in range(rem):
            g = ngrp - rem + r
            b = g % NB
            wait_loads(b)
            fire_gather(b)
            drain_gather((b + 3) % NB)
            fire_scatter((b + 3) % NB)
            drain_scatter((b + 2) % NB)
        bl = (ngrp - 1) % NB
        drain_gather(bl)
        fire_scatter(bl)
        drain_scatter((bl + 3) % NB)
        drain_scatter(bl)
        plsc.subcore_barrier()
        _row_copy(acc, outp.at[cid], sid, n)

    return k2


def _dotT(a, b):
    return lax.dot_general(a, b, (((1,), (1,)), ((), ())),
                           preferred_element_type=F32)


def _k3_body(se_ref, ss_ref, dg_ref, h_ref, tw_ref, iw_ref, tb_ref, ib_ref,
             out_ref):
    deg = dg_ref[...]
    degc = jnp.maximum(deg, 1.0)
    h_o_r = h_ref[...] * se_ref[...] / degc
    num = _dotT(ss_ref[...], tw_ref[...]) + deg * tb_ref[...]
    h_o_s = num / degc
    out_ref[...] = (_dotT(h_o_s, iw_ref[:, :128])
                    + _dotT(h_o_r, iw_ref[:, 128:])
                    + ib_ref[...])


def _make_k3(n):
    bn = 5000
    grid = (n // bn,)
    return pl.pallas_call(
        _k3_body,
        grid=grid,
        in_specs=[
            pl.BlockSpec((bn, 128), lambda i: (i, 0)),
            pl.BlockSpec((bn, 128), lambda i: (i, 0)),
            pl.BlockSpec((bn, 1), lambda i: (i, 0)),
            pl.BlockSpec((bn, 128), lambda i: (i, 0)),
            pl.BlockSpec((128, 128), lambda i: (0, 0)),
            pl.BlockSpec((128, 256), lambda i: (0, 0)),
            pl.BlockSpec((1, 128), lambda i: (0, 0)),
            pl.BlockSpec((1, 128), lambda i: (0, 0)),
        ],
        out_specs=pl.BlockSpec((bn, 128), lambda i: (i, 0)),
        out_shape=jax.ShapeDtypeStruct((n, 128), F32),
    )


def _k4_body(eh_ref, sh_ref, rw_ref, tw_ref, rb_ref, tb_ref, out_ref,
             wc_ref, bc_ref):
    @pl.when(pl.program_id(0) == 0)
    def _():
        rw2 = rw_ref[:, 128:]
        wc_ref[:128, :] = jnp.transpose(rw_ref[:, :128])
        wc_ref[128:, :] = lax.dot_general(tw_ref[...], rw2,
                                          (((0,), (1,)), ((), ())),
                                          preferred_element_type=F32)
        bc_ref[...] = rb_ref[...] + _dotT(tb_ref[...], rw2)

    x = jnp.concatenate([eh_ref[...], sh_ref[...]], axis=1)
    out_ref[...] = (jnp.dot(x, wc_ref[...], preferred_element_type=F32)
                    + bc_ref[...])


def _make_k4(e):
    be = 10000
    grid = (e // be,)
    return pl.pallas_call(
        _k4_body,
        grid=grid,
        in_specs=[
            pl.BlockSpec((be, 128), lambda i: (i, 0)),
            pl.BlockSpec((be, 128), lambda i: (i, 0)),
            pl.BlockSpec((128, 256), lambda i: (0, 0)),
            pl.BlockSpec((128, 128), lambda i: (0, 0)),
            pl.BlockSpec((1, 128), lambda i: (0, 0)),
            pl.BlockSpec((1, 128), lambda i: (0, 0)),
        ],
        out_specs=pl.BlockSpec((be, 128), lambda i: (i, 0)),
        out_shape=jax.ShapeDtypeStruct((e, 128), F32),
        scratch_shapes=[
            pltpu.VMEM((256, 128), F32),
            pltpu.VMEM((1, 128), F32),
        ],
    )


def _k5_body(pp_ref, nm_ref, bv_ref, out_ref):
    out_ref[...] = (pp_ref[0] + pp_ref[1]) * nm_ref[...] + bv_ref[...]


def _make_k5(n):
    bn = 5000
    grid = (n // bn,)
    return pl.pallas_call(
        _k5_body,
        grid=grid,
        in_specs=[
            pl.BlockSpec((2, bn, 128), lambda i: (0, i, 0)),
            pl.BlockSpec((bn, 1), lambda i: (i, 0)),
            pl.BlockSpec((1, 128), lambda i: (0, 0)),
        ],
        out_specs=pl.BlockSpec((bn, 128), lambda i: (i, 0)),
        out_shape=jax.ShapeDtypeStruct((n, 128), F32),
    )


def kernel(h, norm, e_h, s_h, edge_index, text_W, text_b, inv_W, inv_b,
           rel_W, rel_b, bias_v):
    n = h.shape[0]
    e = e_h.shape[0]
    src = edge_index[0]
    dst = edge_index[1]
    tb = text_b.reshape(1, 128)
    ib = inv_b.reshape(1, 128)
    rb = rel_b.reshape(1, 128)
    bv = bias_v.reshape(1, 128)

    e_h_new = _make_k4(e)(e_h, s_h, rel_W, text_W, rb, tb)
    sum_e, sum_s, deg = _make_k1(n, e)(dst, e_h, s_h)
    h_s_r_o = _make_k3(n)(sum_e, sum_s, deg.reshape(n, 1), h, text_W,
                          inv_W, tb, ib)
    (presum,) = _make_k2(n, e)(src, dst, h_s_r_o)
    h_new = _make_k5(n)(presum, norm, bv)
    return h_new, e_h_new
